# Initial kernel scaffold; baseline (speedup 1.0000x reference)
import functools

import jax
import jax.numpy as jnp
from jax import lax
from jax.experimental import pallas as pl
from jax.experimental.pallas import tpu as pltpu
from jax.experimental.pallas import tpu_sc as plsc

N = 50000
HID = 128
ROWS = 51200
TPR = 3200
EPAD = 819200
LPAD = 102400
FG = 32
NB = 2000
LB = 4096

_F32 = jnp.float32


def _mesh():
    return plsc.VectorSubcoreMesh(
        core_axis_name="c", subcore_axis_name="s", num_cores=2, num_subcores=16
    )


def _tc0_body(x_ref, w_ref, b_ref, out_ref):
    x0 = x_ref[:, 0:1]
    x1 = x_ref[:, 1:2]
    p0 = jnp.maximum(x0 * w_ref[0:1, 0:1] + x1 * w_ref[0:1, 1:2] + b_ref[0:1, 0:1], 0.0)
    p1 = jnp.maximum(x0 * w_ref[1:2, 0:1] + x1 * w_ref[1:2, 1:2] + b_ref[0:1, 1:2], 0.0)
    out_ref[:, 0:1] = p0
    out_ref[:, 1:2] = p1
    out_ref[:, 2:3] = jnp.ones_like(p0)
    out_ref[:, 3:4] = jnp.zeros_like(p0)


def _tc0(x, w0p, b0p):
    return pl.pallas_call(
        _tc0_body,
        grid=(N // NB,),
        in_specs=[
            pl.BlockSpec((NB, 2), lambda i: (i, 0)),
            pl.BlockSpec((2, 2), lambda i: (0, 0)),
            pl.BlockSpec((1, 2), lambda i: (0, 0)),
        ],
        out_specs=pl.BlockSpec((NB, 4), lambda i: (i, 0)),
        out_shape=jax.ShapeDtypeStruct((N, 4), _F32),
    )(x, w0p, b0p)


def _layer_norm_rows(h0, g_ref, b_ref):
    mu = jnp.mean(h0, axis=1, keepdims=True)
    var = jnp.mean((h0 - mu) * (h0 - mu), axis=1, keepdims=True)
    return (h0 - mu) * lax.rsqrt(var + 1e-5) * g_ref[0:1, :] + b_ref[0:1, :]


def _tc1_body(p_ref, x_ref, w0lt, w0rt, b0l, g0, beta0, w1pt, b1p,
              h_ref, hpg_ref, cnt_ref):
    ssum = p_ref[0] + p_ref[1]
    cnt = jnp.maximum(ssum[:, 2:3], 1.0)
    m0 = ssum[:, 0:1] / cnt
    m1 = ssum[:, 1:2] / cnt
    x0 = x_ref[:, 0:1]
    x1 = x_ref[:, 1:2]
    h0 = (m0 * w0lt[0:1, :] + m1 * w0lt[1:2, :]
          + x0 * w0rt[0:1, :] + x1 * w0rt[1:2, :] + b0l[0:1, :])
    h = _layer_norm_rows(h0, g0, beta0)
    h_ref[...] = h
    cnt_ref[...] = cnt
    hp = jnp.maximum(
        jnp.dot(h, w1pt[...], preferred_element_type=_F32) + b1p[0:1, :], 0.0)
    for g in range(4):
        hpg_ref[g] = hp[:, g * FG:(g + 1) * FG]


def _tc1(partials, x, w0lt, w0rt, b0l, g0, beta0, w1pt, b1p):
    full = lambda r, c: pl.BlockSpec((r, c), lambda i: (0, 0))
    return pl.pallas_call(
        _tc1_body,
        grid=(N // NB,),
        in_specs=[
            pl.BlockSpec((2, NB, 4), lambda i: (0, i, 0)),
            pl.BlockSpec((NB, 2), lambda i: (i, 0)),
            full(2, HID), full(2, HID), full(1, HID), full(1, HID),
            full(1, HID), full(HID, HID), full(1, HID),
        ],
        out_specs=[
            pl.BlockSpec((NB, HID), lambda i: (i, 0)),
            pl.BlockSpec((4, NB, FG), lambda i: (0, i, 0)),
            pl.BlockSpec((NB, 1), lambda i: (i, 0)),
        ],
        out_shape=[
            jax.ShapeDtypeStruct((N, HID), _F32),
            jax.ShapeDtypeStruct((4, ROWS, FG), _F32),
            jax.ShapeDtypeStruct((N, 1), _F32),
        ],
    )(partials, x, w0lt, w0rt, b0l, g0, beta0, w1pt, b1p)


def _tc2_body(sm_ref, cnt_ref, h_ref, w1lt, b1l, w1rt, g1, beta1,
              we1at, we1bt, ha_ref, hb_ref):
    summed = jnp.concatenate([sm_ref[g] for g in range(4)], axis=1)
    mean1 = summed / cnt_ref[...]
    h1 = (jnp.dot(mean1, w1lt[...], preferred_element_type=_F32)
          + jnp.dot(h_ref[...], w1rt[...], preferred_element_type=_F32)
          + b1l[0:1, :])
    h2 = _layer_norm_rows(h1, g1, beta1)
    ha_ref[...] = jnp.dot(h2, we1at[...], preferred_element_type=_F32)
    hb_ref[...] = jnp.dot(h2, we1bt[...], preferred_element_type=_F32)


def _tc2(summed, cnt, h, w1lt, b1l, w1rt, g1, beta1, we1at, we1bt):
    full = lambda r, c: pl.BlockSpec((r, c), lambda i: (0, 0))
    return pl.pallas_call(
        _tc2_body,
        grid=(N // NB,),
        in_specs=[
            pl.BlockSpec((4, NB, FG), lambda i: (0, i, 0)),
            pl.BlockSpec((NB, 1), lambda i: (i, 0)),
            pl.BlockSpec((NB, HID), lambda i: (i, 0)),
            full(HID, HID), full(1, HID), full(HID, HID), full(1, HID),
            full(1, HID), full(HID, HID), full(HID, HID),
        ],
        out_specs=[
            pl.BlockSpec((NB, HID), lambda i: (i, 0)),
            pl.BlockSpec((NB, HID), lambda i: (i, 0)),
        ],
        out_shape=[
            jax.ShapeDtypeStruct((N, HID), _F32),
            jax.ShapeDtypeStruct((N, HID), _F32),
        ],
    )(summed, cnt, h, w1lt, b1l, w1rt, g1, beta1, we1at, we1bt)


def _tc3_body(s_ref, be1, we2, be2, out_ref):
    t = jnp.maximum(s_ref[...] + be1[0:1, :], 0.0)
    out_ref[...] = jnp.sum(t * we2[0:1, :], axis=1, keepdims=True) + be2[0:1, :]


def _tc3(s, be1, we2, be2):
    full = lambda r, c: pl.BlockSpec((r, c), lambda i: (0, 0))
    return pl.pallas_call(
        _tc3_body,
        grid=(LPAD // LB,),
        in_specs=[
            pl.BlockSpec((LB, HID), lambda i: (i, 0)),
            full(1, HID), full(1, HID), full(1, 1),
        ],
        out_specs=pl.BlockSpec((LB, 1), lambda i: (i, 0)),
        out_shape=jax.ShapeDtypeStruct((LPAD, 1), _F32),
    )(s, be1, we2, be2)


def _gather_scatter_block(table_hbm, acc, sblk, dblk, rows, sems, nrows):
    cps = [None, None]
    cps[0] = pltpu.async_copy(table_hbm.at[sblk.at[0]], rows.at[0], sems[0])
    for r in range(nrows):
        b = r & 1
        cps[b].wait()
        if r + 1 < nrows:
            nb = (r + 1) & 1
            cps[nb] = pltpu.async_copy(table_hbm.at[sblk.at[r + 1]], rows.at[nb], sems[nb])
        pltpu.sync_copy(rows.at[b], acc.at[dblk.at[r]], add=True)


@functools.lru_cache(maxsize=None)
def _sc0_agg():
    @functools.partial(
        pl.kernel,
        out_type=jax.ShapeDtypeStruct((2, ROWS, 4), _F32),
        mesh=_mesh(),
        scratch_types=[
            pltpu.VMEM((8, 128), jnp.int32),
            pltpu.VMEM((8, 128), jnp.int32),
            pltpu.VMEM((2, 128, 4), _F32),
            pltpu.VMEM_SHARED((ROWS, 4), _F32),
            pltpu.SemaphoreType.DMA,
            pltpu.SemaphoreType.DMA,
        ],
    )
    def body(src_hbm, dst_hbm, xp4_hbm, zeros_hbm, out_hbm,
             sblk, dblk, rows, acc, sem0, sem1):
        c = lax.axis_index("c")
        s = lax.axis_index("s")
        row0 = s * TPR
        pltpu.sync_copy(zeros_hbm, acc.at[pl.ds(row0, TPR)])
        plsc.subcore_barrier()

        def blk(b, carry):
            r0 = c * 3200 + s * 200 + b * 8
            pltpu.sync_copy(src_hbm.at[pl.ds(r0, 8)], sblk)
            pltpu.sync_copy(dst_hbm.at[pl.ds(r0, 8)], dblk)
            _gather_scatter_block(xp4_hbm, acc, sblk, dblk, rows, (sem0, sem1), 8)
            return carry

        lax.fori_loop(0, 25, blk, 0)
        plsc.subcore_barrier()
        pltpu.sync_copy(acc.at[pl.ds(row0, TPR)], out_hbm.at[c].at[pl.ds(row0, TPR)])

    return body


@functools.lru_cache(maxsize=None)
def _sc1_agg():
    @functools.partial(
        pl.kernel,
        out_type=jax.ShapeDtypeStruct((4, ROWS, FG), _F32),
        mesh=_mesh(),
        scratch_types=[
            pltpu.VMEM((8, 128), jnp.int32),
            pltpu.VMEM((8, 128), jnp.int32),
            pltpu.VMEM((2, 128, FG), _F32),
            pltpu.VMEM_SHARED((ROWS, FG), _F32),
            pltpu.SemaphoreType.DMA,
            pltpu.SemaphoreType.DMA,
        ],
    )
    def body(src_hbm, dst_hbm, table_hbm, zeros_hbm, out_hbm,
             sblk, dblk, rows, acc, sem0, sem1):
        c = lax.axis_index("c")
        s = lax.axis_index("s")
        row0 = s * TPR
        for gi in range(2):
            g = 2 * c + gi
            off = g * ROWS
            pltpu.sync_copy(zeros_hbm, acc.at[pl.ds(row0, TPR)])
            plsc.subcore_barrier()

            def blk(b, carry):
                r0 = s * 400 + b * 8
                pltpu.sync_copy(src_hbm.at[pl.ds(r0, 8)], sblk)
                pltpu.sync_copy(dst_hbm.at[pl.ds(r0, 8)], dblk)
                for r in range(8):
                    for i in range(8):
                        sl = pl.ds(i * 16, 16)
                        sblk[r, sl] = sblk[r, sl] + off
                _gather_scatter_block(table_hbm, acc, sblk, dblk, rows, (sem0, sem1), 8)
                return carry

            lax.fori_loop(0, 50, blk, 0)
            plsc.subcore_barrier()
            pltpu.sync_copy(acc.at[pl.ds(row0, TPR)], out_hbm.at[g].at[pl.ds(row0, TPR)])
            plsc.subcore_barrier()

    return body


@functools.lru_cache(maxsize=None)
def _sc2_gather():
    @functools.partial(
        pl.kernel,
        out_type=jax.ShapeDtypeStruct((LPAD, HID), _F32),
        mesh=_mesh(),
        scratch_types=[
            pltpu.VMEM((25, 128), jnp.int32),
            pltpu.VMEM((25, 128), jnp.int32),
            pltpu.VMEM((2, 128, HID), _F32),
            pltpu.VMEM((2, 128, HID), _F32),
            pltpu.VMEM((128, HID), _F32),
            pltpu.SemaphoreType.DMA,
            pltpu.SemaphoreType.DMA,
            pltpu.SemaphoreType.DMA,
            pltpu.SemaphoreType.DMA,
        ],
    )
    def body(ha_hbm, hb_hbm, sidx_hbm, didx_hbm, out_hbm,
             sblk, dblk, bufa, bufb, outb, sa0, sa1, sb0, sb1):
        c = lax.axis_index("c")
        s = lax.axis_index("s")
        wid = s * 2 + c
        base = wid * 25
        pltpu.sync_copy(sidx_hbm.at[pl.ds(base, 25)], sblk)
        pltpu.sync_copy(didx_hbm.at[pl.ds(base, 25)], dblk)
        sems_a = (sa0, sa1)
        sems_b = (sb0, sb1)
        cpa = [None, None]
        cpb = [None, None]
        cpa[0] = pltpu.async_copy(ha_hbm.at[sblk.at[0]], bufa.at[0], sems_a[0])
        cpb[0] = pltpu.async_copy(hb_hbm.at[dblk.at[0]], bufb.at[0], sems_b[0])
        for j in range(25):
            b = j & 1
            cpa[b].wait()
            cpb[b].wait()
            if j + 1 < 25:
                nb = (j + 1) & 1
                cpa[nb] = pltpu.async_copy(ha_hbm.at[sblk.at[j + 1]], bufa.at[nb], sems_a[nb])
                cpb[nb] = pltpu.async_copy(hb_hbm.at[dblk.at[j + 1]], bufb.at[nb], sems_b[nb])

            def row_add(i, carry):
                for k in range(HID // 16):
                    sl = pl.ds(k * 16, 16)
                    outb[i, sl] = bufa[b, i, sl] + bufb[b, i, sl]
                return carry

            lax.fori_loop(0, 128, row_add, 0)
            pltpu.sync_copy(outb, out_hbm.at[pl.ds((base + j) * 128, 128)])

    return body


def kernel(x, edge_index, edge_label_index, W0p, b0p, W0l, b0l, W0r, g0, beta0,
           W1p, b1p, W1l, b1l, W1r, g1, beta1, We1, be1, We2, be2):
    E = edge_index.shape[1]
    L = edge_label_index.shape[1]
    i32 = jnp.int32

    src = jnp.concatenate([edge_index[0], jnp.zeros((EPAD - E,), i32)])
    dst = jnp.concatenate([edge_index[1], jnp.full((EPAD - E,), N, i32)])
    src2d = src.reshape(EPAD // 128, 128)
    dst2d = dst.reshape(EPAD // 128, 128)
    sidx = jnp.concatenate([edge_label_index[0], jnp.zeros((LPAD - L,), i32)])
    didx = jnp.concatenate([edge_label_index[1], jnp.zeros((LPAD - L,), i32)])
    sidx2d = sidx.reshape(LPAD // 128, 128)
    didx2d = didx.reshape(LPAD // 128, 128)

    xp4 = _tc0(x, W0p, b0p.reshape(1, 2))
    partials = _sc0_agg()(src2d, dst2d, xp4, jnp.zeros((TPR, 4), _F32))
    h, hpg, cnt = _tc1(
        partials, x, W0l.T, W0r.T, b0l.reshape(1, HID), g0.reshape(1, HID),
        beta0.reshape(1, HID), W1p.T, b1p.reshape(1, HID))
    summed1 = _sc1_agg()(src2d, dst2d, hpg.reshape(4 * ROWS, FG),
                         jnp.zeros((TPR, FG), _F32))
    ha, hb = _tc2(
        summed1, cnt, h, W1l.T, b1l.reshape(1, HID), W1r.T, g1.reshape(1, HID),
        beta1.reshape(1, HID), We1[:, :HID].T, We1[:, HID:].T)
    s_feats = _sc2_gather()(ha, hb, sidx2d, didx2d)
    logits = _tc3(s_feats, be1.reshape(1, HID), We2.reshape(1, HID),
                  be2.reshape(1, 1))
    return logits[:L, 0]


# speedup vs baseline: 4.0897x; 4.0897x over previous
import functools

import jax
import jax.numpy as jnp
from jax import lax
from jax.experimental import pallas as pl
from jax.experimental.pallas import tpu as pltpu
from jax.experimental.pallas import tpu_sc as plsc

N = 50000
HID = 128
W0C = 16
ROWS = 51200
TPR = 3200
EPAD = 819200
LPAD = 102400
FG = 32
NB = 2000
LB = 4096

_F32 = jnp.float32


def _mesh():
    return plsc.VectorSubcoreMesh(
        core_axis_name="c", subcore_axis_name="s", num_cores=2, num_subcores=16
    )


_SC_PARAMS = pltpu.CompilerParams(use_tc_tiling_on_sc=False)


def _tc0_body(x_ref, w_ref, b_ref, out_ref):
    x0 = x_ref[:, 0:1]
    x1 = x_ref[:, 1:2]
    p0 = jnp.maximum(x0 * w_ref[0:1, 0:1] + x1 * w_ref[0:1, 1:2] + b_ref[0:1, 0:1], 0.0)
    p1 = jnp.maximum(x0 * w_ref[1:2, 0:1] + x1 * w_ref[1:2, 1:2] + b_ref[0:1, 1:2], 0.0)
    out_ref[:, 0:1] = p0
    out_ref[:, 1:2] = p1
    out_ref[:, 2:3] = jnp.ones_like(p0)
    out_ref[:, 3:] = jnp.zeros((p0.shape[0], W0C - 3), _F32)


def _tc0(x, w0p, b0p):
    return pl.pallas_call(
        _tc0_body,
        grid=(N // NB,),
        in_specs=[
            pl.BlockSpec((NB, 2), lambda i: (i, 0)),
            pl.BlockSpec((2, 2), lambda i: (0, 0)),
            pl.BlockSpec((1, 2), lambda i: (0, 0)),
        ],
        out_specs=pl.BlockSpec((NB, W0C), lambda i: (i, 0)),
        out_shape=jax.ShapeDtypeStruct((N, W0C), _F32),
    )(x, w0p, b0p)


def _layer_norm_rows(h0, g_ref, b_ref):
    mu = jnp.mean(h0, axis=1, keepdims=True)
    var = jnp.mean((h0 - mu) * (h0 - mu), axis=1, keepdims=True)
    return (h0 - mu) / jnp.sqrt(var + 1e-5) * g_ref[0:1, :] + b_ref[0:1, :]


def _tc1_body(p_ref, x_ref, w0lt, w0rt, b0l, g0, beta0, w1pt, b1p,
              h_ref, hpg_ref, cnt_ref):
    ssum = p_ref[0] + p_ref[1]
    cnt = jnp.maximum(ssum[:, 2:3], 1.0)
    m0 = ssum[:, 0:1] / cnt
    m1 = ssum[:, 1:2] / cnt
    x0 = x_ref[:, 0:1]
    x1 = x_ref[:, 1:2]
    h0 = (m0 * w0lt[0:1, :] + m1 * w0lt[1:2, :]
          + x0 * w0rt[0:1, :] + x1 * w0rt[1:2, :] + b0l[0:1, :])
    h = _layer_norm_rows(h0, g0, beta0)
    h_ref[...] = h
    cnt_ref[...] = cnt
    hp = jnp.maximum(
        jnp.dot(h, w1pt[...], preferred_element_type=_F32, precision=lax.Precision.HIGHEST) + b1p[0:1, :], 0.0)
    for g in range(4):
        hpg_ref[g] = hp[:, g * FG:(g + 1) * FG]


def _tc1(partials, x, w0lt, w0rt, b0l, g0, beta0, w1pt, b1p):
    full = lambda r, c: pl.BlockSpec((r, c), lambda i: (0, 0))
    return pl.pallas_call(
        _tc1_body,
        grid=(N // NB,),
        in_specs=[
            pl.BlockSpec((2, NB, W0C), lambda i: (0, i, 0)),
            pl.BlockSpec((NB, 2), lambda i: (i, 0)),
            full(2, HID), full(2, HID), full(1, HID), full(1, HID),
            full(1, HID), full(HID, HID), full(1, HID),
        ],
        out_specs=[
            pl.BlockSpec((NB, HID), lambda i: (i, 0)),
            pl.BlockSpec((4, NB, FG), lambda i: (0, i, 0)),
            pl.BlockSpec((NB, 1), lambda i: (i, 0)),
        ],
        out_shape=[
            jax.ShapeDtypeStruct((N, HID), _F32),
            jax.ShapeDtypeStruct((4, ROWS, FG), _F32),
            jax.ShapeDtypeStruct((N, 1), _F32),
        ],
    )(partials, x, w0lt, w0rt, b0l, g0, beta0, w1pt, b1p)


def _tc2_body(sm_ref, cnt_ref, h_ref, w1lt, b1l, w1rt, g1, beta1,
              we1at, we1bt, ha_ref, hb_ref):
    summed = jnp.concatenate([sm_ref[g] for g in range(4)], axis=1)
    mean1 = summed / cnt_ref[...]
    h1 = (jnp.dot(mean1, w1lt[...], preferred_element_type=_F32, precision=lax.Precision.HIGHEST)
          + jnp.dot(h_ref[...], w1rt[...], preferred_element_type=_F32, precision=lax.Precision.HIGHEST)
          + b1l[0:1, :])
    h2 = _layer_norm_rows(h1, g1, beta1)
    ha_ref[...] = jnp.dot(h2, we1at[...], preferred_element_type=_F32, precision=lax.Precision.HIGHEST)
    hb_ref[...] = jnp.dot(h2, we1bt[...], preferred_element_type=_F32, precision=lax.Precision.HIGHEST)


def _tc2(summed, cnt, h, w1lt, b1l, w1rt, g1, beta1, we1at, we1bt):
    full = lambda r, c: pl.BlockSpec((r, c), lambda i: (0, 0))
    return pl.pallas_call(
        _tc2_body,
        grid=(N // NB,),
        in_specs=[
            pl.BlockSpec((4, NB, FG), lambda i: (0, i, 0)),
            pl.BlockSpec((NB, 1), lambda i: (i, 0)),
            pl.BlockSpec((NB, HID), lambda i: (i, 0)),
            full(HID, HID), full(1, HID), full(HID, HID), full(1, HID),
            full(1, HID), full(HID, HID), full(HID, HID),
        ],
        out_specs=[
            pl.BlockSpec((NB, HID), lambda i: (i, 0)),
            pl.BlockSpec((NB, HID), lambda i: (i, 0)),
        ],
        out_shape=[
            jax.ShapeDtypeStruct((N, HID), _F32),
            jax.ShapeDtypeStruct((N, HID), _F32),
        ],
    )(summed, cnt, h, w1lt, b1l, w1rt, g1, beta1, we1at, we1bt)


def _tc3_body(s_ref, be1, we2, be2, out_ref):
    t = jnp.maximum(s_ref[...] + be1[0:1, :], 0.0)
    out_ref[...] = jnp.sum(t * we2[0:1, :], axis=1, keepdims=True) + be2[0:1, :]


def _tc3(s, be1, we2, be2):
    full = lambda r, c: pl.BlockSpec((r, c), lambda i: (0, 0))
    return pl.pallas_call(
        _tc3_body,
        grid=(LPAD // LB,),
        in_specs=[
            pl.BlockSpec((LB, HID), lambda i: (i, 0)),
            full(1, HID), full(1, HID), full(1, 1),
        ],
        out_specs=pl.BlockSpec((LB, 1), lambda i: (i, 0)),
        out_shape=jax.ShapeDtypeStruct((LPAD, 1), _F32),
    )(s, be1, we2, be2)


def _gather_scatter_block(table_hbm, acc, sblk, dblk, rows, sems, nrows):
    cps = [None, None]
    cps[0] = pltpu.async_copy(table_hbm.at[sblk.at[0]], rows.at[0], sems[0])
    for r in range(nrows):
        b = r & 1
        cps[b].wait()
        if r + 1 < nrows:
            nb = (r + 1) & 1
            cps[nb] = pltpu.async_copy(table_hbm.at[sblk.at[r + 1]], rows.at[nb], sems[nb])
        pltpu.sync_copy(rows.at[b], acc.at[dblk.at[r]], add=True)


@functools.lru_cache(maxsize=None)
def _sc0_agg():
    @functools.partial(
        pl.kernel,
        out_type=jax.ShapeDtypeStruct((2, ROWS, W0C), _F32),
        mesh=_mesh(),
        compiler_params=_SC_PARAMS,
        scratch_types=[
            pltpu.VMEM((8, 128), jnp.int32),
            pltpu.VMEM((8, 128), jnp.int32),
            pltpu.VMEM((2, 128, W0C), _F32),
            pltpu.VMEM_SHARED((ROWS, W0C), _F32),
            pltpu.SemaphoreType.DMA,
            pltpu.SemaphoreType.DMA,
        ],
    )
    def body(src_hbm, dst_hbm, xp4_hbm, zeros_hbm, out_hbm,
             sblk, dblk, rows, acc, sem0, sem1):
        c = lax.axis_index("c")
        s = lax.axis_index("s")
        row0 = s * TPR
        pltpu.sync_copy(zeros_hbm, acc.at[pl.ds(row0, TPR)])
        plsc.subcore_barrier()

        def blk(b, carry):
            r0 = c * 3200 + s * 200 + b * 8
            pltpu.sync_copy(src_hbm.at[pl.ds(r0, 8)], sblk)
            pltpu.sync_copy(dst_hbm.at[pl.ds(r0, 8)], dblk)
            _gather_scatter_block(xp4_hbm, acc, sblk, dblk, rows, (sem0, sem1), 8)
            return carry

        lax.fori_loop(0, 25, blk, 0)
        plsc.subcore_barrier()
        pltpu.sync_copy(acc.at[pl.ds(row0, TPR)], out_hbm.at[c].at[pl.ds(row0, TPR)])

    return body


@functools.lru_cache(maxsize=None)
def _sc1_agg():
    @functools.partial(
        pl.kernel,
        out_type=jax.ShapeDtypeStruct((4, ROWS, FG), _F32),
        mesh=_mesh(),
        compiler_params=_SC_PARAMS,
        scratch_types=[
            pltpu.VMEM((8, 128), jnp.int32),
            pltpu.VMEM((8, 128), jnp.int32),
            pltpu.VMEM((2, 128, FG), _F32),
            pltpu.VMEM_SHARED((ROWS, FG), _F32),
            pltpu.SemaphoreType.DMA,
            pltpu.SemaphoreType.DMA,
        ],
    )
    def body(src_hbm, dst_hbm, table_hbm, zeros_hbm, out_hbm,
             sblk, dblk, rows, acc, sem0, sem1):
        c = lax.axis_index("c")
        s = lax.axis_index("s")
        row0 = s * TPR
        for gi in range(2):
            g = 2 * c + gi
            off = g * ROWS
            pltpu.sync_copy(zeros_hbm, acc.at[pl.ds(row0, TPR)])
            plsc.subcore_barrier()

            def blk(b, carry):
                r0 = s * 400 + b * 8
                pltpu.sync_copy(src_hbm.at[pl.ds(r0, 8)], sblk)
                pltpu.sync_copy(dst_hbm.at[pl.ds(r0, 8)], dblk)
                for r in range(8):
                    for i in range(8):
                        sl = pl.ds(i * 16, 16)
                        sblk[r, sl] = sblk[r, sl] + off
                _gather_scatter_block(table_hbm, acc, sblk, dblk, rows, (sem0, sem1), 8)
                return carry

            lax.fori_loop(0, 50, blk, 0)
            plsc.subcore_barrier()
            pltpu.sync_copy(acc.at[pl.ds(row0, TPR)], out_hbm.at[g].at[pl.ds(row0, TPR)])
            plsc.subcore_barrier()

    return body


@functools.lru_cache(maxsize=None)
def _sc2_gather():
    @functools.partial(
        pl.kernel,
        out_type=jax.ShapeDtypeStruct((LPAD, HID), _F32),
        mesh=_mesh(),
        compiler_params=_SC_PARAMS,
        scratch_types=[
            pltpu.VMEM((25, 128), jnp.int32),
            pltpu.VMEM((25, 128), jnp.int32),
            pltpu.VMEM((2, 128, HID), _F32),
            pltpu.VMEM((2, 128, HID), _F32),
            pltpu.VMEM((128, HID), _F32),
            pltpu.SemaphoreType.DMA,
            pltpu.SemaphoreType.DMA,
            pltpu.SemaphoreType.DMA,
            pltpu.SemaphoreType.DMA,
        ],
    )
    def body(ha_hbm, hb_hbm, sidx_hbm, didx_hbm, out_hbm,
             sblk, dblk, bufa, bufb, outb, sa0, sa1, sb0, sb1):
        c = lax.axis_index("c")
        s = lax.axis_index("s")
        wid = s * 2 + c
        base = wid * 25
        pltpu.sync_copy(sidx_hbm.at[pl.ds(base, 25)], sblk)
        pltpu.sync_copy(didx_hbm.at[pl.ds(base, 25)], dblk)
        sems_a = (sa0, sa1)
        sems_b = (sb0, sb1)
        cpa = [None, None]
        cpb = [None, None]
        cpa[0] = pltpu.async_copy(ha_hbm.at[sblk.at[0]], bufa.at[0], sems_a[0])
        cpb[0] = pltpu.async_copy(hb_hbm.at[dblk.at[0]], bufb.at[0], sems_b[0])
        for j in range(25):
            b = j & 1
            cpa[b].wait()
            cpb[b].wait()
            if j + 1 < 25:
                nb = (j + 1) & 1
                cpa[nb] = pltpu.async_copy(ha_hbm.at[sblk.at[j + 1]], bufa.at[nb], sems_a[nb])
                cpb[nb] = pltpu.async_copy(hb_hbm.at[dblk.at[j + 1]], bufb.at[nb], sems_b[nb])

            def row_add(i, carry):
                for k in range(HID // 16):
                    sl = pl.ds(k * 16, 16)
                    outb[i, sl] = bufa[b, i, sl] + bufb[b, i, sl]
                return carry

            lax.fori_loop(0, 128, row_add, 0)
            pltpu.sync_copy(outb, out_hbm.at[pl.ds((base + j) * 128, 128)])

    return body


def kernel(x, edge_index, edge_label_index, W0p, b0p, W0l, b0l, W0r, g0, beta0,
           W1p, b1p, W1l, b1l, W1r, g1, beta1, We1, be1, We2, be2):
    E = edge_index.shape[1]
    L = edge_label_index.shape[1]
    i32 = jnp.int32

    src = jnp.concatenate([edge_index[0], jnp.zeros((EPAD - E,), i32)])
    dst = jnp.concatenate([edge_index[1], jnp.full((EPAD - E,), N, i32)])
    src2d = src.reshape(EPAD // 128, 128)
    dst2d = dst.reshape(EPAD // 128, 128)
    sidx = jnp.concatenate([edge_label_index[0], jnp.zeros((LPAD - L,), i32)])
    didx = jnp.concatenate([edge_label_index[1], jnp.zeros((LPAD - L,), i32)])
    sidx2d = sidx.reshape(LPAD // 128, 128)
    didx2d = didx.reshape(LPAD // 128, 128)

    xp4 = _tc0(x, W0p, b0p.reshape(1, 2))
    partials = _sc0_agg()(src2d, dst2d, xp4, jnp.zeros((TPR, W0C), _F32))
    h, hpg, cnt = _tc1(
        partials, x, W0l.T, W0r.T, b0l.reshape(1, HID), g0.reshape(1, HID),
        beta0.reshape(1, HID), W1p.T, b1p.reshape(1, HID))
    summed1 = _sc1_agg()(src2d, dst2d, hpg.reshape(4 * ROWS, FG),
                         jnp.zeros((TPR, FG), _F32))
    ha, hb = _tc2(
        summed1, cnt, h, W1l.T, b1l.reshape(1, HID), W1r.T, g1.reshape(1, HID),
        beta1.reshape(1, HID), We1[:, :HID].T, We1[:, HID:].T)
    s_feats = _sc2_gather()(ha, hb, sidx2d, didx2d)
    logits = _tc3(s_feats, be1.reshape(1, HID), We2.reshape(1, HID),
                  be2.reshape(1, 1))
    return logits[:L, 0]


# speedup vs baseline: 4.4526x; 1.0887x over previous
import functools

import jax
import jax.numpy as jnp
from jax import lax
from jax.experimental import pallas as pl
from jax.experimental.pallas import tpu as pltpu
from jax.experimental.pallas import tpu_sc as plsc

N = 50000
HID = 128
W0C = 16
ROWS = 51200
TPR = 3200
EPAD = 819200
LPAD = 102400
FG = 16
NG = 8
NB = 2000
LB = 4096

_F32 = jnp.float32


def _mesh():
    return plsc.VectorSubcoreMesh(
        core_axis_name="c", subcore_axis_name="s", num_cores=2, num_subcores=16
    )


_SC_PARAMS = pltpu.CompilerParams(use_tc_tiling_on_sc=False)


def _tc0_body(x_ref, w_ref, b_ref, out_ref):
    x0 = x_ref[:, 0:1]
    x1 = x_ref[:, 1:2]
    p0 = jnp.maximum(x0 * w_ref[0:1, 0:1] + x1 * w_ref[0:1, 1:2] + b_ref[0:1, 0:1], 0.0)
    p1 = jnp.maximum(x0 * w_ref[1:2, 0:1] + x1 * w_ref[1:2, 1:2] + b_ref[0:1, 1:2], 0.0)
    out_ref[:, 0:1] = p0
    out_ref[:, 1:2] = p1
    out_ref[:, 2:3] = jnp.ones_like(p0)
    out_ref[:, 3:] = jnp.zeros((p0.shape[0], W0C - 3), _F32)


def _tc0(x, w0p, b0p):
    return pl.pallas_call(
        _tc0_body,
        grid=(N // NB,),
        in_specs=[
            pl.BlockSpec((NB, 2), lambda i: (i, 0)),
            pl.BlockSpec((2, 2), lambda i: (0, 0)),
            pl.BlockSpec((1, 2), lambda i: (0, 0)),
        ],
        out_specs=pl.BlockSpec((NB, W0C), lambda i: (i, 0)),
        out_shape=jax.ShapeDtypeStruct((N, W0C), _F32),
    )(x, w0p, b0p)


def _layer_norm_rows(h0, g_ref, b_ref):
    mu = jnp.mean(h0, axis=1, keepdims=True)
    var = jnp.mean((h0 - mu) * (h0 - mu), axis=1, keepdims=True)
    return (h0 - mu) / jnp.sqrt(var + 1e-5) * g_ref[0:1, :] + b_ref[0:1, :]


def _tc1_body(p_ref, x_ref, w0lt, w0rt, b0l, g0, beta0, w1pt, b1p,
              h_ref, hpg_ref, cnt_ref):
    ssum = p_ref[0] + p_ref[1]
    cnt = jnp.maximum(ssum[:, 2:3], 1.0)
    m0 = ssum[:, 0:1] / cnt
    m1 = ssum[:, 1:2] / cnt
    x0 = x_ref[:, 0:1]
    x1 = x_ref[:, 1:2]
    h0 = (m0 * w0lt[0:1, :] + m1 * w0lt[1:2, :]
          + x0 * w0rt[0:1, :] + x1 * w0rt[1:2, :] + b0l[0:1, :])
    h = _layer_norm_rows(h0, g0, beta0)
    h_ref[...] = h
    cnt_ref[...] = cnt
    hp = jnp.maximum(
        jnp.dot(h, w1pt[...], preferred_element_type=_F32, precision=lax.Precision.HIGHEST) + b1p[0:1, :], 0.0)
    for g in range(NG):
        hpg_ref[g] = hp[:, g * FG:(g + 1) * FG]


def _tc1(partials, x, w0lt, w0rt, b0l, g0, beta0, w1pt, b1p):
    full = lambda r, c: pl.BlockSpec((r, c), lambda i: (0, 0))
    return pl.pallas_call(
        _tc1_body,
        grid=(N // NB,),
        in_specs=[
            pl.BlockSpec((2, NB, W0C), lambda i: (0, i, 0)),
            pl.BlockSpec((NB, 2), lambda i: (i, 0)),
            full(2, HID), full(2, HID), full(1, HID), full(1, HID),
            full(1, HID), full(HID, HID), full(1, HID),
        ],
        out_specs=[
            pl.BlockSpec((NB, HID), lambda i: (i, 0)),
            pl.BlockSpec((NG, NB, FG), lambda i: (0, i, 0)),
            pl.BlockSpec((NB, 1), lambda i: (i, 0)),
        ],
        out_shape=[
            jax.ShapeDtypeStruct((N, HID), _F32),
            jax.ShapeDtypeStruct((NG, ROWS, FG), _F32),
            jax.ShapeDtypeStruct((N, 1), _F32),
        ],
    )(partials, x, w0lt, w0rt, b0l, g0, beta0, w1pt, b1p)


def _tc2_body(sm_ref, cnt_ref, h_ref, w1lt, b1l, w1rt, g1, beta1,
              we1at, we1bt, ha_ref, hb_ref):
    summed = jnp.concatenate([sm_ref[g] for g in range(NG)], axis=1)
    mean1 = summed / cnt_ref[...]
    h1 = (jnp.dot(mean1, w1lt[...], preferred_element_type=_F32, precision=lax.Precision.HIGHEST)
          + jnp.dot(h_ref[...], w1rt[...], preferred_element_type=_F32, precision=lax.Precision.HIGHEST)
          + b1l[0:1, :])
    h2 = _layer_norm_rows(h1, g1, beta1)
    ha_ref[...] = jnp.dot(h2, we1at[...], preferred_element_type=_F32, precision=lax.Precision.HIGHEST)
    hb_ref[...] = jnp.dot(h2, we1bt[...], preferred_element_type=_F32, precision=lax.Precision.HIGHEST)


def _tc2(summed, cnt, h, w1lt, b1l, w1rt, g1, beta1, we1at, we1bt):
    full = lambda r, c: pl.BlockSpec((r, c), lambda i: (0, 0))
    return pl.pallas_call(
        _tc2_body,
        grid=(N // NB,),
        in_specs=[
            pl.BlockSpec((NG, NB, FG), lambda i: (0, i, 0)),
            pl.BlockSpec((NB, 1), lambda i: (i, 0)),
            pl.BlockSpec((NB, HID), lambda i: (i, 0)),
            full(HID, HID), full(1, HID), full(HID, HID), full(1, HID),
            full(1, HID), full(HID, HID), full(HID, HID),
        ],
        out_specs=[
            pl.BlockSpec((NB, HID), lambda i: (i, 0)),
            pl.BlockSpec((NB, HID), lambda i: (i, 0)),
        ],
        out_shape=[
            jax.ShapeDtypeStruct((N, HID), _F32),
            jax.ShapeDtypeStruct((N, HID), _F32),
        ],
    )(summed, cnt, h, w1lt, b1l, w1rt, g1, beta1, we1at, we1bt)


def _tc3_body(s_ref, be1, we2, be2, out_ref):
    t = jnp.maximum(s_ref[...] + be1[0:1, :], 0.0)
    out_ref[...] = jnp.sum(t * we2[0:1, :], axis=1, keepdims=True) + be2[0:1, :]


def _tc3(s, be1, we2, be2):
    full = lambda r, c: pl.BlockSpec((r, c), lambda i: (0, 0))
    return pl.pallas_call(
        _tc3_body,
        grid=(LPAD // LB,),
        in_specs=[
            pl.BlockSpec((LB, HID), lambda i: (i, 0)),
            full(1, HID), full(1, HID), full(1, 1),
        ],
        out_specs=pl.BlockSpec((LB, 1), lambda i: (i, 0)),
        out_shape=jax.ShapeDtypeStruct((LPAD, 1), _F32),
    )(s, be1, we2, be2)


SBR = 50
GRP = 10
K5 = 5


def _agg_groups(table, acc, sb_s, sb_d, rows, gs, ss):
    def fire_g(g):
        q = g & 1
        return [pltpu.async_copy(table.at[sb_s.at[g * K5 + r]],
                                 rows.at[q].at[r], gs[q]) for r in range(K5)]

    def fire_s(g):
        q = g & 1
        return [pltpu.async_copy(rows.at[q].at[r],
                                 acc.at[sb_d.at[g * K5 + r]], ss[q], add=True)
                for r in range(K5)]

    scat = [None] * GRP
    gcur = fire_g(0)
    for g in range(GRP):
        for cp in gcur:
            cp.wait()
        if g + 1 < GRP:
            if g >= 1:
                for cp in scat[g - 1]:
                    cp.wait()
            gcur = fire_g(g + 1)
        scat[g] = fire_s(g)
    for cp in scat[GRP - 2]:
        cp.wait()
    for cp in scat[GRP - 1]:
        cp.wait()


def _offset_slab(slab, off):
    def add_row(r, carry):
        for i in range(8):
            sl = pl.ds(i * 16, 16)
            slab[r, sl] = slab[r, sl] + off
        return carry

    lax.fori_loop(0, SBR, add_row, 0)


def _agg_pass(src_hbm, dst_hbm, table, acc, sblk, dblk, rows, isem, gs, ss,
              base, n_sb, off=None):
    pltpu.sync_copy(src_hbm.at[pl.ds(base, SBR)], sblk.at[0])
    pltpu.sync_copy(dst_hbm.at[pl.ds(base, SBR)], dblk.at[0])
    if off is not None:
        _offset_slab(sblk.at[0], off)

    def sb_pair(k, carry):
        for p in range(2):
            sb = 2 * k + p
            nxt = 1 - p
            nxt_row = base + jnp.where(sb + 1 < n_sb, (sb + 1) * SBR, 0)
            icps = [
                pltpu.async_copy(src_hbm.at[pl.ds(nxt_row, SBR)], sblk.at[nxt], isem),
                pltpu.async_copy(dst_hbm.at[pl.ds(nxt_row, SBR)], dblk.at[nxt], isem),
            ]
            _agg_groups(table, acc, sblk.at[p], dblk.at[p], rows, gs, ss)
            for cp in icps:
                cp.wait()
            if off is not None:
                _offset_slab(sblk.at[nxt], off)
        return carry

    lax.fori_loop(0, n_sb // 2, sb_pair, 0)


@functools.lru_cache(maxsize=None)
def _sc0_agg():
    @functools.partial(
        pl.kernel,
        out_type=jax.ShapeDtypeStruct((2, ROWS, W0C), _F32),
        mesh=_mesh(),
        compiler_params=_SC_PARAMS,
        scratch_types=[
            pltpu.VMEM((2, SBR, 128), jnp.int32),
            pltpu.VMEM((2, SBR, 128), jnp.int32),
            pltpu.VMEM((2, K5, 128, W0C), _F32),
            pltpu.VMEM_SHARED((ROWS, W0C), _F32),
            pltpu.SemaphoreType.DMA,
            pltpu.SemaphoreType.DMA,
            pltpu.SemaphoreType.DMA,
            pltpu.SemaphoreType.DMA,
            pltpu.SemaphoreType.DMA,
        ],
    )
    def body(src_hbm, dst_hbm, xp4_hbm, zeros_hbm, out_hbm,
             sblk, dblk, rows, acc, isem, g0s, g1s, s0s, s1s):
        c = lax.axis_index("c")
        s = lax.axis_index("s")
        row0 = s * TPR
        pltpu.sync_copy(zeros_hbm, acc.at[pl.ds(row0, TPR)])
        plsc.subcore_barrier()
        _agg_pass(src_hbm, dst_hbm, xp4_hbm, acc, sblk, dblk, rows,
                  isem, (g0s, g1s), (s0s, s1s), c * 3200 + s * 200, 4)
        plsc.subcore_barrier()
        pltpu.sync_copy(acc.at[pl.ds(row0, TPR)], out_hbm.at[c].at[pl.ds(row0, TPR)])

    return body


@functools.lru_cache(maxsize=None)
def _sc1_agg():
    @functools.partial(
        pl.kernel,
        out_type=jax.ShapeDtypeStruct((NG, ROWS, FG), _F32),
        mesh=_mesh(),
        compiler_params=_SC_PARAMS,
        scratch_types=[
            pltpu.VMEM((2, SBR, 128), jnp.int32),
            pltpu.VMEM((2, SBR, 128), jnp.int32),
            pltpu.VMEM((2, K5, 128, FG), _F32),
            pltpu.VMEM_SHARED((ROWS, FG), _F32),
            pltpu.SemaphoreType.DMA,
            pltpu.SemaphoreType.DMA,
            pltpu.SemaphoreType.DMA,
            pltpu.SemaphoreType.DMA,
            pltpu.SemaphoreType.DMA,
        ],
    )
    def body(src_hbm, dst_hbm, table_hbm, zeros_hbm, out_hbm,
             sblk, dblk, rows, acc, isem, g0s, g1s, s0s, s1s):
        c = lax.axis_index("c")
        s = lax.axis_index("s")
        row0 = s * TPR
        tab = table_hbm.at[c]

        def gi_body(gi, carry):
            g = 4 * c + gi
            pltpu.sync_copy(zeros_hbm, acc.at[pl.ds(row0, TPR)])
            plsc.subcore_barrier()
            _agg_pass(src_hbm, dst_hbm, tab, acc, sblk, dblk, rows,
                      isem, (g0s, g1s), (s0s, s1s), s * 400, 8, off=gi * ROWS)
            plsc.subcore_barrier()
            pltpu.sync_copy(acc.at[pl.ds(row0, TPR)], out_hbm.at[g].at[pl.ds(row0, TPR)])
            plsc.subcore_barrier()
            return carry

        lax.fori_loop(0, 4, gi_body, 0)

    return body


@functools.lru_cache(maxsize=None)
def _sc2_gather():
    @functools.partial(
        pl.kernel,
        out_type=jax.ShapeDtypeStruct((LPAD, HID), _F32),
        mesh=_mesh(),
        compiler_params=_SC_PARAMS,
        scratch_types=[
            pltpu.VMEM((25, 128), jnp.int32),
            pltpu.VMEM((25, 128), jnp.int32),
            pltpu.VMEM((2, 128, HID), _F32),
            pltpu.VMEM((2, 128, HID), _F32),
            pltpu.VMEM((2, 128, HID), _F32),
            pltpu.SemaphoreType.DMA,
            pltpu.SemaphoreType.DMA,
            pltpu.SemaphoreType.DMA,
            pltpu.SemaphoreType.DMA,
            pltpu.SemaphoreType.DMA,
            pltpu.SemaphoreType.DMA,
        ],
    )
    def body(ha_hbm, hb_hbm, sidx_hbm, didx_hbm, out_hbm,
             sblk, dblk, bufa, bufb, outb, sa0, sa1, sb0, sb1, w0, w1):
        c = lax.axis_index("c")
        s = lax.axis_index("s")
        base = (s * 2 + c) * 25
        pltpu.sync_copy(sidx_hbm.at[pl.ds(base, 25)], sblk)
        pltpu.sync_copy(didx_hbm.at[pl.ds(base, 25)], dblk)
        sems_a = (sa0, sa1)
        sems_b = (sb0, sb1)
        sems_w = (w0, w1)
        cpa = [None, None]
        cpb = [None, None]
        cpw = [None, None]
        cpa[0] = pltpu.async_copy(ha_hbm.at[sblk.at[0]], bufa.at[0], sems_a[0])
        cpb[0] = pltpu.async_copy(hb_hbm.at[dblk.at[0]], bufb.at[0], sems_b[0])
        for j in range(25):
            p = j & 1
            cpa[p].wait()
            cpb[p].wait()
            if j + 1 < 25:
                np_ = (j + 1) & 1
                cpa[np_] = pltpu.async_copy(ha_hbm.at[sblk.at[j + 1]], bufa.at[np_], sems_a[np_])
                cpb[np_] = pltpu.async_copy(hb_hbm.at[dblk.at[j + 1]], bufb.at[np_], sems_b[np_])
            if cpw[p] is not None:
                cpw[p].wait()

            def row_add(i, carry):
                for k in range(HID // 16):
                    sl = pl.ds(k * 16, 16)
                    outb[p, i, sl] = bufa[p, i, sl] + bufb[p, i, sl]
                return carry

            lax.fori_loop(0, 128, row_add, 0)
            cpw[p] = pltpu.async_copy(outb.at[p], out_hbm.at[pl.ds((base + j) * 128, 128)], sems_w[p])
        cpw[0].wait()
        cpw[1].wait()

    return body


def kernel(x, edge_index, edge_label_index, W0p, b0p, W0l, b0l, W0r, g0, beta0,
           W1p, b1p, W1l, b1l, W1r, g1, beta1, We1, be1, We2, be2):
    E = edge_index.shape[1]
    L = edge_label_index.shape[1]
    i32 = jnp.int32

    src = jnp.concatenate([edge_index[0], jnp.zeros((EPAD - E,), i32)])
    dst = jnp.concatenate([edge_index[1], jnp.full((EPAD - E,), N, i32)])
    src2d = src.reshape(EPAD // 128, 128)
    dst2d = dst.reshape(EPAD // 128, 128)
    sidx = jnp.concatenate([edge_label_index[0], jnp.zeros((LPAD - L,), i32)])
    didx = jnp.concatenate([edge_label_index[1], jnp.zeros((LPAD - L,), i32)])
    sidx2d = sidx.reshape(LPAD // 128, 128)
    didx2d = didx.reshape(LPAD // 128, 128)

    xp4 = _tc0(x, W0p, b0p.reshape(1, 2))
    partials = _sc0_agg()(src2d, dst2d, xp4, jnp.zeros((TPR, W0C), _F32))
    h, hpg, cnt = _tc1(
        partials, x, W0l.T, W0r.T, b0l.reshape(1, HID), g0.reshape(1, HID),
        beta0.reshape(1, HID), W1p.T, b1p.reshape(1, HID))
    summed1 = _sc1_agg()(src2d, dst2d, hpg.reshape(2, 4 * ROWS, FG),
                         jnp.zeros((TPR, FG), _F32))
    ha, hb = _tc2(
        summed1, cnt, h, W1l.T, b1l.reshape(1, HID), W1r.T, g1.reshape(1, HID),
        beta1.reshape(1, HID), We1[:, :HID].T, We1[:, HID:].T)
    s_feats = _sc2_gather()(ha, hb, sidx2d, didx2d)
    logits = _tc3(s_feats, be1.reshape(1, HID), We2.reshape(1, HID),
                  be2.reshape(1, 1))
    return logits[:L, 0]


# speedup vs baseline: 4.9046x; 1.1015x over previous
import functools

import jax
import jax.numpy as jnp
from jax import lax
from jax.experimental import pallas as pl
from jax.experimental.pallas import tpu as pltpu
from jax.experimental.pallas import tpu_sc as plsc

N = 50000
HID = 128
W0C = 16
ROWS = 51200
TPR = 3200
EPAD = 819200
LPAD = 102400
FG = 32
NG = 4
NB = 2000
LB = 4096

_F32 = jnp.float32


def _mesh():
    return plsc.VectorSubcoreMesh(
        core_axis_name="c", subcore_axis_name="s", num_cores=2, num_subcores=16
    )


_SC_PARAMS = pltpu.CompilerParams(use_tc_tiling_on_sc=False)


def _tc0_body(x_ref, w_ref, b_ref, out_ref):
    x0 = x_ref[:, 0:1]
    x1 = x_ref[:, 1:2]
    p0 = jnp.maximum(x0 * w_ref[0:1, 0:1] + x1 * w_ref[0:1, 1:2] + b_ref[0:1, 0:1], 0.0)
    p1 = jnp.maximum(x0 * w_ref[1:2, 0:1] + x1 * w_ref[1:2, 1:2] + b_ref[0:1, 1:2], 0.0)
    out_ref[:, 0:1] = p0
    out_ref[:, 1:2] = p1
    out_ref[:, 2:3] = jnp.ones_like(p0)
    out_ref[:, 3:] = jnp.zeros((p0.shape[0], W0C - 3), _F32)


def _tc0(x, w0p, b0p):
    return pl.pallas_call(
        _tc0_body,
        grid=(N // NB,),
        in_specs=[
            pl.BlockSpec((NB, 2), lambda i: (i, 0)),
            pl.BlockSpec((2, 2), lambda i: (0, 0)),
            pl.BlockSpec((1, 2), lambda i: (0, 0)),
        ],
        out_specs=pl.BlockSpec((NB, W0C), lambda i: (i, 0)),
        out_shape=jax.ShapeDtypeStruct((N, W0C), _F32),
    )(x, w0p, b0p)


def _layer_norm_rows(h0, g_ref, b_ref):
    mu = jnp.mean(h0, axis=1, keepdims=True)
    var = jnp.mean((h0 - mu) * (h0 - mu), axis=1, keepdims=True)
    return (h0 - mu) / jnp.sqrt(var + 1e-5) * g_ref[0:1, :] + b_ref[0:1, :]


def _tc1_body(p_ref, x_ref, w0lt, w0rt, b0l, g0, beta0, w1pt, b1p,
              h_ref, hpg_ref, cnt_ref):
    ssum = p_ref[0] + p_ref[1]
    cnt = jnp.maximum(ssum[:, 2:3], 1.0)
    m0 = ssum[:, 0:1] / cnt
    m1 = ssum[:, 1:2] / cnt
    x0 = x_ref[:, 0:1]
    x1 = x_ref[:, 1:2]
    h0 = (m0 * w0lt[0:1, :] + m1 * w0lt[1:2, :]
          + x0 * w0rt[0:1, :] + x1 * w0rt[1:2, :] + b0l[0:1, :])
    h = _layer_norm_rows(h0, g0, beta0)
    h_ref[...] = h
    cnt_ref[...] = cnt
    hp = jnp.maximum(
        jnp.dot(h, w1pt[...], preferred_element_type=_F32, precision=lax.Precision.HIGHEST) + b1p[0:1, :], 0.0)
    for g in range(NG):
        hpg_ref[g] = hp[:, g * FG:(g + 1) * FG]


def _tc1(partials, x, w0lt, w0rt, b0l, g0, beta0, w1pt, b1p):
    full = lambda r, c: pl.BlockSpec((r, c), lambda i: (0, 0))
    return pl.pallas_call(
        _tc1_body,
        grid=(N // NB,),
        in_specs=[
            pl.BlockSpec((2, NB, W0C), lambda i: (0, i, 0)),
            pl.BlockSpec((NB, 2), lambda i: (i, 0)),
            full(2, HID), full(2, HID), full(1, HID), full(1, HID),
            full(1, HID), full(HID, HID), full(1, HID),
        ],
        out_specs=[
            pl.BlockSpec((NB, HID), lambda i: (i, 0)),
            pl.BlockSpec((NG, NB, FG), lambda i: (0, i, 0)),
            pl.BlockSpec((NB, 1), lambda i: (i, 0)),
        ],
        out_shape=[
            jax.ShapeDtypeStruct((N, HID), _F32),
            jax.ShapeDtypeStruct((NG, ROWS, FG), _F32),
            jax.ShapeDtypeStruct((N, 1), _F32),
        ],
    )(partials, x, w0lt, w0rt, b0l, g0, beta0, w1pt, b1p)


def _tc2_body(sm_ref, cnt_ref, h_ref, w1lt, b1l, w1rt, g1, beta1,
              we1at, we1bt, ha_ref, hb_ref):
    summed = jnp.concatenate([sm_ref[g] for g in range(NG)], axis=1)
    mean1 = summed / cnt_ref[...]
    h1 = (jnp.dot(mean1, w1lt[...], preferred_element_type=_F32, precision=lax.Precision.HIGHEST)
          + jnp.dot(h_ref[...], w1rt[...], preferred_element_type=_F32, precision=lax.Precision.HIGHEST)
          + b1l[0:1, :])
    h2 = _layer_norm_rows(h1, g1, beta1)
    ha_ref[...] = jnp.dot(h2, we1at[...], preferred_element_type=_F32, precision=lax.Precision.HIGHEST)
    hb_ref[...] = jnp.dot(h2, we1bt[...], preferred_element_type=_F32, precision=lax.Precision.HIGHEST)


def _tc2(summed, cnt, h, w1lt, b1l, w1rt, g1, beta1, we1at, we1bt):
    full = lambda r, c: pl.BlockSpec((r, c), lambda i: (0, 0))
    return pl.pallas_call(
        _tc2_body,
        grid=(N // NB,),
        in_specs=[
            pl.BlockSpec((NG, NB, FG), lambda i: (0, i, 0)),
            pl.BlockSpec((NB, 1), lambda i: (i, 0)),
            pl.BlockSpec((NB, HID), lambda i: (i, 0)),
            full(HID, HID), full(1, HID), full(HID, HID), full(1, HID),
            full(1, HID), full(HID, HID), full(HID, HID),
        ],
        out_specs=[
            pl.BlockSpec((NB, HID), lambda i: (i, 0)),
            pl.BlockSpec((NB, HID), lambda i: (i, 0)),
        ],
        out_shape=[
            jax.ShapeDtypeStruct((N, HID), _F32),
            jax.ShapeDtypeStruct((N, HID), _F32),
        ],
    )(summed, cnt, h, w1lt, b1l, w1rt, g1, beta1, we1at, we1bt)


def _tc3_body(s_ref, be1, we2, be2, out_ref):
    t = jnp.maximum(s_ref[...] + be1[0:1, :], 0.0)
    out_ref[...] = jnp.sum(t * we2[0:1, :], axis=1, keepdims=True) + be2[0:1, :]


def _tc3(s, be1, we2, be2):
    full = lambda r, c: pl.BlockSpec((r, c), lambda i: (0, 0))
    return pl.pallas_call(
        _tc3_body,
        grid=(LPAD // LB,),
        in_specs=[
            pl.BlockSpec((LB, HID), lambda i: (i, 0)),
            full(1, HID), full(1, HID), full(1, 1),
        ],
        out_specs=pl.BlockSpec((LB, 1), lambda i: (i, 0)),
        out_shape=jax.ShapeDtypeStruct((LPAD, 1), _F32),
    )(s, be1, we2, be2)


def _agg_groups(table, acc, sb_s, sb_d, rows, gs, ss, grp, kk):
    def fire_g(g):
        q = g & 1
        return [pltpu.async_copy(table.at[sb_s.at[g * kk + r]],
                                 rows.at[q].at[r], gs[q]) for r in range(kk)]

    def fire_s(g):
        q = g & 1
        return [pltpu.async_copy(rows.at[q].at[r],
                                 acc.at[sb_d.at[g * kk + r]], ss[q], add=True)
                for r in range(kk)]

    scat = [None] * grp
    gcur = fire_g(0)
    for g in range(grp):
        for cp in gcur:
            cp.wait()
        if g + 1 < grp:
            if g >= 1:
                for cp in scat[g - 1]:
                    cp.wait()
            gcur = fire_g(g + 1)
        scat[g] = fire_s(g)
    for cp in scat[grp - 2]:
        cp.wait()
    for cp in scat[grp - 1]:
        cp.wait()


def _offset_slab(slab, off, sbr):
    def add_row(r, carry):
        for i in range(8):
            sl = pl.ds(i * 16, 16)
            slab[r, sl] = slab[r, sl] + off
        return carry

    lax.fori_loop(0, sbr, add_row, 0)


def _agg_pass(src_hbm, dst_hbm, table, acc, sblk, dblk, rows, isem, gs, ss,
              base, sbr, n_sb, grp, kk, off=None):
    pltpu.sync_copy(src_hbm.at[pl.ds(base, sbr)], sblk.at[0])
    pltpu.sync_copy(dst_hbm.at[pl.ds(base, sbr)], dblk.at[0])
    if off is not None:
        _offset_slab(sblk.at[0], off, sbr)

    def sb_pair(k, carry):
        for p in range(2):
            sb = 2 * k + p
            nxt = 1 - p
            nxt_row = base + jnp.where(sb + 1 < n_sb, (sb + 1) * sbr, 0)
            icps = [
                pltpu.async_copy(src_hbm.at[pl.ds(nxt_row, sbr)], sblk.at[nxt], isem),
                pltpu.async_copy(dst_hbm.at[pl.ds(nxt_row, sbr)], dblk.at[nxt], isem),
            ]
            _agg_groups(table, acc, sblk.at[p], dblk.at[p], rows, gs, ss, grp, kk)
            for cp in icps:
                cp.wait()
            if off is not None:
                _offset_slab(sblk.at[nxt], off, sbr)
        return carry

    lax.fori_loop(0, n_sb // 2, sb_pair, 0)


@functools.lru_cache(maxsize=None)
def _sc0_agg():
    @functools.partial(
        pl.kernel,
        out_type=jax.ShapeDtypeStruct((2, ROWS, W0C), _F32),
        mesh=_mesh(),
        compiler_params=_SC_PARAMS,
        scratch_types=[
            pltpu.VMEM((2, 50, 128), jnp.int32),
            pltpu.VMEM((2, 50, 128), jnp.int32),
            pltpu.VMEM((2, 5, 128, W0C), _F32),
            pltpu.VMEM_SHARED((ROWS, W0C), _F32),
            pltpu.SemaphoreType.DMA,
            pltpu.SemaphoreType.DMA,
            pltpu.SemaphoreType.DMA,
            pltpu.SemaphoreType.DMA,
            pltpu.SemaphoreType.DMA,
        ],
    )
    def body(src_hbm, dst_hbm, xp4_hbm, zeros_hbm, out_hbm,
             sblk, dblk, rows, acc, isem, g0s, g1s, s0s, s1s):
        c = lax.axis_index("c")
        s = lax.axis_index("s")
        row0 = s * TPR
        pltpu.sync_copy(zeros_hbm, acc.at[pl.ds(row0, TPR)])
        plsc.subcore_barrier()
        _agg_pass(src_hbm, dst_hbm, xp4_hbm, acc, sblk, dblk, rows,
                  isem, (g0s, g1s), (s0s, s1s), c * 3200 + s * 200,
                  50, 4, 10, 5)
        plsc.subcore_barrier()
        pltpu.sync_copy(acc.at[pl.ds(row0, TPR)], out_hbm.at[c].at[pl.ds(row0, TPR)])

    return body


@functools.lru_cache(maxsize=None)
def _sc1_agg():
    @functools.partial(
        pl.kernel,
        out_type=jax.ShapeDtypeStruct((NG, ROWS, FG), _F32),
        mesh=_mesh(),
        compiler_params=_SC_PARAMS,
        scratch_types=[
            pltpu.VMEM((2, 20, 128), jnp.int32),
            pltpu.VMEM((2, 20, 128), jnp.int32),
            pltpu.VMEM((2, 2, 128, FG), _F32),
            pltpu.VMEM_SHARED((ROWS, FG), _F32),
            pltpu.SemaphoreType.DMA,
            pltpu.SemaphoreType.DMA,
            pltpu.SemaphoreType.DMA,
            pltpu.SemaphoreType.DMA,
            pltpu.SemaphoreType.DMA,
        ],
    )
    def body(src_hbm, dst_hbm, table_hbm, zeros_hbm, out_hbm,
             sblk, dblk, rows, acc, isem, g0s, g1s, s0s, s1s):
        c = lax.axis_index("c")
        s = lax.axis_index("s")
        row0 = s * TPR
        tab = table_hbm.at[c]

        def gi_body(gi, carry):
            g = (NG // 2) * c + gi
            pltpu.sync_copy(zeros_hbm, acc.at[pl.ds(row0, TPR)])
            plsc.subcore_barrier()
            _agg_pass(src_hbm, dst_hbm, tab, acc, sblk, dblk, rows,
                      isem, (g0s, g1s), (s0s, s1s), s * 400,
                      20, 20, 10, 2, off=gi * ROWS)
            plsc.subcore_barrier()
            pltpu.sync_copy(acc.at[pl.ds(row0, TPR)], out_hbm.at[g].at[pl.ds(row0, TPR)])
            plsc.subcore_barrier()
            return carry

        lax.fori_loop(0, NG // 2, gi_body, 0)

    return body


@functools.lru_cache(maxsize=None)
def _sc2_gather():
    @functools.partial(
        pl.kernel,
        out_type=jax.ShapeDtypeStruct((LPAD, HID), _F32),
        mesh=_mesh(),
        compiler_params=_SC_PARAMS,
        scratch_types=[
            pltpu.VMEM((25, 128), jnp.int32),
            pltpu.VMEM((25, 128), jnp.int32),
            pltpu.VMEM((2, 128, HID), _F32),
            pltpu.VMEM((2, 128, HID), _F32),
            pltpu.VMEM((2, 128, HID), _F32),
            pltpu.SemaphoreType.DMA,
            pltpu.SemaphoreType.DMA,
            pltpu.SemaphoreType.DMA,
            pltpu.SemaphoreType.DMA,
            pltpu.SemaphoreType.DMA,
            pltpu.SemaphoreType.DMA,
        ],
    )
    def body(ha_hbm, hb_hbm, sidx_hbm, didx_hbm, out_hbm,
             sblk, dblk, bufa, bufb, outb, sa0, sa1, sb0, sb1, w0, w1):
        c = lax.axis_index("c")
        s = lax.axis_index("s")
        base = (s * 2 + c) * 25
        pltpu.sync_copy(sidx_hbm.at[pl.ds(base, 25)], sblk)
        pltpu.sync_copy(didx_hbm.at[pl.ds(base, 25)], dblk)
        sems_a = (sa0, sa1)
        sems_b = (sb0, sb1)
        sems_w = (w0, w1)
        cpa = [None, None]
        cpb = [None, None]
        cpw = [None, None]
        cpa[0] = pltpu.async_copy(ha_hbm.at[sblk.at[0]], bufa.at[0], sems_a[0])
        cpb[0] = pltpu.async_copy(hb_hbm.at[dblk.at[0]], bufb.at[0], sems_b[0])
        for j in range(25):
            p = j & 1
            cpa[p].wait()
            cpb[p].wait()
            if j + 1 < 25:
                np_ = (j + 1) & 1
                cpa[np_] = pltpu.async_copy(ha_hbm.at[sblk.at[j + 1]], bufa.at[np_], sems_a[np_])
                cpb[np_] = pltpu.async_copy(hb_hbm.at[dblk.at[j + 1]], bufb.at[np_], sems_b[np_])
            if cpw[p] is not None:
                cpw[p].wait()

            def row_add(i, carry):
                for k in range(HID // 16):
                    sl = pl.ds(k * 16, 16)
                    outb[p, i, sl] = bufa[p, i, sl] + bufb[p, i, sl]
                return carry

            lax.fori_loop(0, 128, row_add, 0)
            cpw[p] = pltpu.async_copy(outb.at[p], out_hbm.at[pl.ds((base + j) * 128, 128)], sems_w[p])
        cpw[0].wait()
        cpw[1].wait()

    return body


def kernel(x, edge_index, edge_label_index, W0p, b0p, W0l, b0l, W0r, g0, beta0,
           W1p, b1p, W1l, b1l, W1r, g1, beta1, We1, be1, We2, be2):
    E = edge_index.shape[1]
    L = edge_label_index.shape[1]
    i32 = jnp.int32

    src = jnp.concatenate([edge_index[0], jnp.zeros((EPAD - E,), i32)])
    dst = jnp.concatenate([edge_index[1], jnp.full((EPAD - E,), N, i32)])
    src2d = src.reshape(EPAD // 128, 128)
    dst2d = dst.reshape(EPAD // 128, 128)
    sidx = jnp.concatenate([edge_label_index[0], jnp.zeros((LPAD - L,), i32)])
    didx = jnp.concatenate([edge_label_index[1], jnp.zeros((LPAD - L,), i32)])
    sidx2d = sidx.reshape(LPAD // 128, 128)
    didx2d = didx.reshape(LPAD // 128, 128)

    xp4 = _tc0(x, W0p, b0p.reshape(1, 2))
    partials = _sc0_agg()(src2d, dst2d, xp4, jnp.zeros((TPR, W0C), _F32))
    h, hpg, cnt = _tc1(
        partials, x, W0l.T, W0r.T, b0l.reshape(1, HID), g0.reshape(1, HID),
        beta0.reshape(1, HID), W1p.T, b1p.reshape(1, HID))
    summed1 = _sc1_agg()(src2d, dst2d, hpg.reshape(2, (NG // 2) * ROWS, FG),
                         jnp.zeros((TPR, FG), _F32))
    ha, hb = _tc2(
        summed1, cnt, h, W1l.T, b1l.reshape(1, HID), W1r.T, g1.reshape(1, HID),
        beta1.reshape(1, HID), We1[:, :HID].T, We1[:, HID:].T)
    s_feats = _sc2_gather()(ha, hb, sidx2d, didx2d)
    logits = _tc3(s_feats, be1.reshape(1, HID), We2.reshape(1, HID),
                  be2.reshape(1, 1))
    return logits[:L, 0]


# speedup vs baseline: 4.9922x; 1.0179x over previous
import functools

import jax
import jax.numpy as jnp
from jax import lax
from jax.experimental import pallas as pl
from jax.experimental.pallas import tpu as pltpu
from jax.experimental.pallas import tpu_sc as plsc

N = 50000
HID = 128
W0C = 16
ROWS = 51200
TPR = 3200
EPAD = 819200
LPAD = 102400
FG = 32
NG = 4
NB = 2000
LB = 4096

_F32 = jnp.float32
TAB0 = 50048


def _mesh():
    return plsc.VectorSubcoreMesh(
        core_axis_name="c", subcore_axis_name="s", num_cores=2, num_subcores=16
    )


_SC_PARAMS = pltpu.CompilerParams(use_tc_tiling_on_sc=False)


def _tc0_body(x_ref, w_ref, b_ref, out_ref):
    x0 = x_ref[:, 0:1]
    x1 = x_ref[:, 1:2]
    p0 = jnp.maximum(x0 * w_ref[0:1, 0:1] + x1 * w_ref[0:1, 1:2] + b_ref[0:1, 0:1], 0.0)
    p1 = jnp.maximum(x0 * w_ref[1:2, 0:1] + x1 * w_ref[1:2, 1:2] + b_ref[0:1, 1:2], 0.0)
    out_ref[:, 0:1] = p0
    out_ref[:, 1:2] = p1
    out_ref[:, 2:3] = jnp.ones_like(p0)
    out_ref[:, 3:] = jnp.zeros((p0.shape[0], W0C - 3), _F32)


def _tc0(x, w0p, b0p):
    return pl.pallas_call(
        _tc0_body,
        grid=(N // NB,),
        in_specs=[
            pl.BlockSpec((NB, 2), lambda i: (i, 0)),
            pl.BlockSpec((2, 2), lambda i: (0, 0)),
            pl.BlockSpec((1, 2), lambda i: (0, 0)),
        ],
        out_specs=pl.BlockSpec((NB, W0C), lambda i: (i, 0)),
        out_shape=jax.ShapeDtypeStruct((TAB0, W0C), _F32),
    )(x, w0p, b0p)


def _layer_norm_rows(h0, g_ref, b_ref):
    mu = jnp.mean(h0, axis=1, keepdims=True)
    var = jnp.mean((h0 - mu) * (h0 - mu), axis=1, keepdims=True)
    return (h0 - mu) / jnp.sqrt(var + 1e-5) * g_ref[0:1, :] + b_ref[0:1, :]


def _tc1_body(p_ref, x_ref, w0lt, w0rt, b0l, g0, beta0, w1pt, b1p,
              h_ref, hpg_ref, cnt_ref):
    ssum = p_ref[0] + p_ref[1]
    cnt = jnp.maximum(ssum[:, 2:3], 1.0)
    m0 = ssum[:, 0:1] / cnt
    m1 = ssum[:, 1:2] / cnt
    x0 = x_ref[:, 0:1]
    x1 = x_ref[:, 1:2]
    h0 = (m0 * w0lt[0:1, :] + m1 * w0lt[1:2, :]
          + x0 * w0rt[0:1, :] + x1 * w0rt[1:2, :] + b0l[0:1, :])
    h = _layer_norm_rows(h0, g0, beta0)
    h_ref[...] = h
    cnt_ref[...] = cnt
    hpg_ref[...] = jnp.maximum(
        jnp.dot(h, w1pt[...], preferred_element_type=_F32, precision=lax.Precision.HIGHEST) + b1p[0:1, :], 0.0)


def _tc1(partials, x, w0lt, w0rt, b0l, g0, beta0, w1pt, b1p):
    full = lambda r, c: pl.BlockSpec((r, c), lambda i: (0, 0))
    return pl.pallas_call(
        _tc1_body,
        grid=(N // NB,),
        in_specs=[
            pl.BlockSpec((2, NB, W0C), lambda i: (0, i, 0)),
            pl.BlockSpec((NB, 2), lambda i: (i, 0)),
            full(2, HID), full(2, HID), full(1, HID), full(1, HID),
            full(1, HID), full(HID, HID), full(1, HID),
        ],
        out_specs=[
            pl.BlockSpec((NB, HID), lambda i: (i, 0)),
            pl.BlockSpec((NB, HID), lambda i: (i, 0)),
            pl.BlockSpec((NB, 1), lambda i: (i, 0)),
        ],
        out_shape=[
            jax.ShapeDtypeStruct((N, HID), _F32),
            jax.ShapeDtypeStruct((N, HID), _F32),
            jax.ShapeDtypeStruct((N, 1), _F32),
        ],
    )(partials, x, w0lt, w0rt, b0l, g0, beta0, w1pt, b1p)


def _tc2_body(sm_ref, cnt_ref, h_ref, w1lt, b1l, w1rt, g1, beta1,
              we1at, we1bt, ha_ref, hb_ref):
    summed = jnp.concatenate([sm_ref[g] for g in range(NG)], axis=1)
    mean1 = summed / cnt_ref[...]
    h1 = (jnp.dot(mean1, w1lt[...], preferred_element_type=_F32, precision=lax.Precision.HIGHEST)
          + jnp.dot(h_ref[...], w1rt[...], preferred_element_type=_F32, precision=lax.Precision.HIGHEST)
          + b1l[0:1, :])
    h2 = _layer_norm_rows(h1, g1, beta1)
    ha_ref[...] = jnp.dot(h2, we1at[...], preferred_element_type=_F32, precision=lax.Precision.HIGHEST)
    hb_ref[...] = jnp.dot(h2, we1bt[...], preferred_element_type=_F32, precision=lax.Precision.HIGHEST)


def _tc2(summed, cnt, h, w1lt, b1l, w1rt, g1, beta1, we1at, we1bt):
    full = lambda r, c: pl.BlockSpec((r, c), lambda i: (0, 0))
    return pl.pallas_call(
        _tc2_body,
        grid=(N // NB,),
        in_specs=[
            pl.BlockSpec((NG, NB, FG), lambda i: (0, i, 0)),
            pl.BlockSpec((NB, 1), lambda i: (i, 0)),
            pl.BlockSpec((NB, HID), lambda i: (i, 0)),
            full(HID, HID), full(1, HID), full(HID, HID), full(1, HID),
            full(1, HID), full(HID, HID), full(HID, HID),
        ],
        out_specs=[
            pl.BlockSpec((NB, HID), lambda i: (i, 0)),
            pl.BlockSpec((NB, HID), lambda i: (i, 0)),
        ],
        out_shape=[
            jax.ShapeDtypeStruct((N, HID), _F32),
            jax.ShapeDtypeStruct((N, HID), _F32),
        ],
    )(summed, cnt, h, w1lt, b1l, w1rt, g1, beta1, we1at, we1bt)


def _tc3_body(s_ref, be1, we2, be2, out_ref):
    t = jnp.maximum(s_ref[...] + be1[0:1, :], 0.0)
    out_ref[...] = jnp.sum(t * we2[0:1, :], axis=1, keepdims=True) + be2[0:1, :]


def _tc3(s, be1, we2, be2):
    full = lambda r, c: pl.BlockSpec((r, c), lambda i: (0, 0))
    return pl.pallas_call(
        _tc3_body,
        grid=(LPAD // LB,),
        in_specs=[
            pl.BlockSpec((LB, HID), lambda i: (i, 0)),
            full(1, HID), full(1, HID), full(1, 1),
        ],
        out_specs=pl.BlockSpec((LB, 1), lambda i: (i, 0)),
        out_shape=jax.ShapeDtypeStruct((LPAD, 1), _F32),
    )(s, be1, we2, be2)


def _agg_groups(table, acc, sb_s, sb_d, rows, gs, ss, grp, kk):
    def fire_g(g):
        q = g & 1
        return [pltpu.async_copy(table.at[sb_s.at[g * kk + r]],
                                 rows.at[q].at[r], gs[q]) for r in range(kk)]

    def fire_s(g):
        q = g & 1
        return [pltpu.async_copy(rows.at[q].at[r],
                                 acc.at[sb_d.at[g * kk + r]], ss[q], add=True)
                for r in range(kk)]

    scat = [None] * grp
    gcur = fire_g(0)
    for g in range(grp):
        for cp in gcur:
            cp.wait()
        if g + 1 < grp:
            if g >= 1:
                for cp in scat[g - 1]:
                    cp.wait()
            gcur = fire_g(g + 1)
        scat[g] = fire_s(g)
    for cp in scat[grp - 2]:
        cp.wait()
    for cp in scat[grp - 1]:
        cp.wait()


def _offset_slab(slab, off, sbr):
    def add_row(r, carry):
        for i in range(8):
            sl = pl.ds(i * 16, 16)
            slab[r, sl] = slab[r, sl] * NG + off
        return carry

    lax.fori_loop(0, sbr, add_row, 0)


def _agg_pass(src_hbm, dst_hbm, table, acc, sblk, dblk, rows, isem, gs, ss,
              base, sbr, n_sb, grp, kk, off=None):
    pltpu.sync_copy(src_hbm.at[pl.ds(base, sbr)], sblk.at[0])
    pltpu.sync_copy(dst_hbm.at[pl.ds(base, sbr)], dblk.at[0])
    if off is not None:
        _offset_slab(sblk.at[0], off, sbr)

    def sb_pair(k, carry):
        for p in range(2):
            sb = 2 * k + p
            nxt = 1 - p
            nxt_row = base + jnp.where(sb + 1 < n_sb, (sb + 1) * sbr, 0)
            icps = [
                pltpu.async_copy(src_hbm.at[pl.ds(nxt_row, sbr)], sblk.at[nxt], isem),
                pltpu.async_copy(dst_hbm.at[pl.ds(nxt_row, sbr)], dblk.at[nxt], isem),
            ]
            _agg_groups(table, acc, sblk.at[p], dblk.at[p], rows, gs, ss, grp, kk)
            for cp in icps:
                cp.wait()
            if off is not None:
                _offset_slab(sblk.at[nxt], off, sbr)
        return carry

    lax.fori_loop(0, n_sb // 2, sb_pair, 0)


@functools.lru_cache(maxsize=None)
def _sc0_agg():
    @functools.partial(
        pl.kernel,
        out_type=jax.ShapeDtypeStruct((2, ROWS, W0C), _F32),
        mesh=_mesh(),
        compiler_params=_SC_PARAMS,
        scratch_types=[
            pltpu.VMEM((2, 20, 128), jnp.int32),
            pltpu.VMEM((2, 20, 128), jnp.int32),
            pltpu.VMEM((2, 2, 128, W0C), _F32),
            pltpu.VMEM_SHARED((ROWS, W0C), _F32),
            pltpu.VMEM_SHARED((TAB0, W0C), _F32),
            pltpu.SemaphoreType.DMA,
            pltpu.SemaphoreType.DMA,
            pltpu.SemaphoreType.DMA,
            pltpu.SemaphoreType.DMA,
            pltpu.SemaphoreType.DMA,
        ],
    )
    def body(src_hbm, dst_hbm, xp4_hbm, zeros_hbm, out_hbm,
             sblk, dblk, rows, acc, tab, isem, g0s, g1s, s0s, s1s):
        c = lax.axis_index("c")
        s = lax.axis_index("s")
        row0 = s * TPR
        pltpu.sync_copy(xp4_hbm.at[pl.ds(s * (TAB0 // 16), TAB0 // 16)],
                        tab.at[pl.ds(s * (TAB0 // 16), TAB0 // 16)])
        pltpu.sync_copy(zeros_hbm, acc.at[pl.ds(row0, TPR)])
        plsc.subcore_barrier()
        _agg_pass(src_hbm, dst_hbm, tab, acc, sblk, dblk, rows,
                  isem, (g0s, g1s), (s0s, s1s), c * 3200 + s * 200,
                  20, 10, 10, 2)
        plsc.subcore_barrier()
        pltpu.sync_copy(acc.at[pl.ds(row0, TPR)], out_hbm.at[c].at[pl.ds(row0, TPR)])

    return body


@functools.lru_cache(maxsize=None)
def _sc1_agg():
    @functools.partial(
        pl.kernel,
        out_type=jax.ShapeDtypeStruct((NG, ROWS, FG), _F32),
        mesh=_mesh(),
        compiler_params=_SC_PARAMS,
        scratch_types=[
            pltpu.VMEM((2, 20, 128), jnp.int32),
            pltpu.VMEM((2, 20, 128), jnp.int32),
            pltpu.VMEM((2, 2, 128, FG), _F32),
            pltpu.VMEM_SHARED((ROWS, FG), _F32),
            pltpu.SemaphoreType.DMA,
            pltpu.SemaphoreType.DMA,
            pltpu.SemaphoreType.DMA,
            pltpu.SemaphoreType.DMA,
            pltpu.SemaphoreType.DMA,
        ],
    )
    def body(src_hbm, dst_hbm, table_hbm, zeros_hbm, out_hbm,
             sblk, dblk, rows, acc, isem, g0s, g1s, s0s, s1s):
        c = lax.axis_index("c")
        s = lax.axis_index("s")
        row0 = s * TPR

        def gi_body(gi, carry):
            g = (NG // 2) * c + gi
            pltpu.sync_copy(zeros_hbm, acc.at[pl.ds(row0, TPR)])
            plsc.subcore_barrier()
            _agg_pass(src_hbm, dst_hbm, table_hbm, acc, sblk, dblk, rows,
                      isem, (g0s, g1s), (s0s, s1s), s * 400,
                      20, 20, 10, 2, off=g)
            plsc.subcore_barrier()
            pltpu.sync_copy(acc.at[pl.ds(row0, TPR)], out_hbm.at[g].at[pl.ds(row0, TPR)])
            plsc.subcore_barrier()
            return carry

        lax.fori_loop(0, NG // 2, gi_body, 0)

    return body


@functools.lru_cache(maxsize=None)
def _sc2_gather():
    @functools.partial(
        pl.kernel,
        out_type=jax.ShapeDtypeStruct((LPAD, HID), _F32),
        mesh=_mesh(),
        compiler_params=_SC_PARAMS,
        scratch_types=[
            pltpu.VMEM((25, 128), jnp.int32),
            pltpu.VMEM((25, 128), jnp.int32),
            pltpu.VMEM((2, 128, HID), _F32),
            pltpu.VMEM((2, 128, HID), _F32),
            pltpu.VMEM((2, 128, HID), _F32),
            pltpu.SemaphoreType.DMA,
            pltpu.SemaphoreType.DMA,
            pltpu.SemaphoreType.DMA,
            pltpu.SemaphoreType.DMA,
            pltpu.SemaphoreType.DMA,
            pltpu.SemaphoreType.DMA,
        ],
    )
    def body(ha_hbm, hb_hbm, sidx_hbm, didx_hbm, out_hbm,
             sblk, dblk, bufa, bufb, outb, sa0, sa1, sb0, sb1, w0, w1):
        c = lax.axis_index("c")
        s = lax.axis_index("s")
        base = (c * 16 + s) * 25
        pltpu.sync_copy(sidx_hbm.at[pl.ds(base, 25)], sblk)
        pltpu.sync_copy(didx_hbm.at[pl.ds(base, 25)], dblk)
        sems_a = (sa0, sa1)
        sems_b = (sb0, sb1)
        sems_w = (w0, w1)
        cpa = [None, None]
        cpb = [None, None]
        cpw = [None, None]
        cpa[0] = pltpu.async_copy(ha_hbm.at[sblk.at[0]], bufa.at[0], sems_a[0])
        cpb[0] = pltpu.async_copy(hb_hbm.at[dblk.at[0]], bufb.at[0], sems_b[0])
        for j in range(25):
            p = j & 1
            cpa[p].wait()
            cpb[p].wait()
            if j + 1 < 25:
                np_ = (j + 1) & 1
                cpa[np_] = pltpu.async_copy(ha_hbm.at[sblk.at[j + 1]], bufa.at[np_], sems_a[np_])
                cpb[np_] = pltpu.async_copy(hb_hbm.at[dblk.at[j + 1]], bufb.at[np_], sems_b[np_])
            if cpw[p] is not None:
                cpw[p].wait()

            def row_add(i, carry):
                for k in range(HID // 16):
                    sl = pl.ds(k * 16, 16)
                    outb[p, i, sl] = bufa[p, i, sl] + bufb[p, i, sl]
                return carry

            lax.fori_loop(0, 128, row_add, 0)
            cpw[p] = pltpu.async_copy(outb.at[p], out_hbm.at[pl.ds((base + j) * 128, 128)], sems_w[p])
        cpw[0].wait()
        cpw[1].wait()

    return body


def kernel(x, edge_index, edge_label_index, W0p, b0p, W0l, b0l, W0r, g0, beta0,
           W1p, b1p, W1l, b1l, W1r, g1, beta1, We1, be1, We2, be2):
    E = edge_index.shape[1]
    L = edge_label_index.shape[1]
    i32 = jnp.int32

    src = jnp.concatenate([edge_index[0], jnp.zeros((EPAD - E,), i32)])
    dst = jnp.concatenate([edge_index[1], jnp.full((EPAD - E,), N, i32)])
    src2d = src.reshape(EPAD // 128, 128)
    dst2d = dst.reshape(EPAD // 128, 128)
    sidx = jnp.concatenate([edge_label_index[0], jnp.zeros((LPAD - L,), i32)])
    didx = jnp.concatenate([edge_label_index[1], jnp.zeros((LPAD - L,), i32)])
    sidx2d = sidx.reshape(LPAD // 128, 128)
    didx2d = didx.reshape(LPAD // 128, 128)

    xp4 = _tc0(x, W0p, b0p.reshape(1, 2))
    partials = _sc0_agg()(src2d, dst2d, xp4, jnp.zeros((TPR, W0C), _F32))
    h, hpg, cnt = _tc1(
        partials, x, W0l.T, W0r.T, b0l.reshape(1, HID), g0.reshape(1, HID),
        beta0.reshape(1, HID), W1p.T, b1p.reshape(1, HID))
    summed1 = _sc1_agg()(src2d, dst2d, hpg.reshape(N * NG, FG),
                         jnp.zeros((TPR, FG), _F32))
    ha, hb = _tc2(
        summed1, cnt, h, W1l.T, b1l.reshape(1, HID), W1r.T, g1.reshape(1, HID),
        beta1.reshape(1, HID), We1[:, :HID].T, We1[:, HID:].T)
    s_feats = _sc2_gather()(ha, hb, sidx2d, didx2d)
    logits = _tc3(s_feats, be1.reshape(1, HID), We2.reshape(1, HID),
                  be2.reshape(1, 1))
    return logits[:L, 0]


# speedup vs baseline: 5.6251x; 1.1268x over previous
import functools

import jax
import jax.numpy as jnp
from jax import lax
from jax.experimental import pallas as pl
from jax.experimental.pallas import tpu as pltpu
from jax.experimental.pallas import tpu_sc as plsc

N = 50000
HID = 128
W0C = 16
ROWS = 51200
TPR = 3200
EPAD = 819200
LPAD = 102400
FG = 32
NG = 4
NB = 2000
LB = 4096

_F32 = jnp.float32
TAB0 = 50048


def _mesh():
    return plsc.VectorSubcoreMesh(
        core_axis_name="c", subcore_axis_name="s", num_cores=2, num_subcores=16
    )


_SC_PARAMS = pltpu.CompilerParams(use_tc_tiling_on_sc=False)


def _tc0_body(x_ref, w_ref, b_ref, out_ref):
    x0 = x_ref[:, 0:1]
    x1 = x_ref[:, 1:2]
    p0 = jnp.maximum(x0 * w_ref[0:1, 0:1] + x1 * w_ref[0:1, 1:2] + b_ref[0:1, 0:1], 0.0)
    p1 = jnp.maximum(x0 * w_ref[1:2, 0:1] + x1 * w_ref[1:2, 1:2] + b_ref[0:1, 1:2], 0.0)
    out_ref[:, 0:1] = p0
    out_ref[:, 1:2] = p1
    out_ref[:, 2:3] = jnp.ones_like(p0)
    out_ref[:, 3:] = jnp.zeros((p0.shape[0], W0C - 3), _F32)


def _tc0(x, w0p, b0p):
    return pl.pallas_call(
        _tc0_body,
        grid=(N // NB,),
        in_specs=[
            pl.BlockSpec((NB, 2), lambda i: (i, 0)),
            pl.BlockSpec((2, 2), lambda i: (0, 0)),
            pl.BlockSpec((1, 2), lambda i: (0, 0)),
        ],
        out_specs=pl.BlockSpec((NB, W0C), lambda i: (i, 0)),
        out_shape=jax.ShapeDtypeStruct((TAB0, W0C), _F32),
    )(x, w0p, b0p)


def _layer_norm_rows(h0, g_ref, b_ref):
    mu = jnp.mean(h0, axis=1, keepdims=True)
    var = jnp.mean((h0 - mu) * (h0 - mu), axis=1, keepdims=True)
    return (h0 - mu) / jnp.sqrt(var + 1e-5) * g_ref[0:1, :] + b_ref[0:1, :]


def _tc1_body(p_ref, x_ref, w0lt, w0rt, b0l, g0, beta0, w1pt, b1p,
              h_ref, hpg_ref, cnt_ref):
    ssum = p_ref[0] + p_ref[1]
    cnt = jnp.maximum(ssum[:, 2:3], 1.0)
    m0 = ssum[:, 0:1] / cnt
    m1 = ssum[:, 1:2] / cnt
    x0 = x_ref[:, 0:1]
    x1 = x_ref[:, 1:2]
    h0 = (m0 * w0lt[0:1, :] + m1 * w0lt[1:2, :]
          + x0 * w0rt[0:1, :] + x1 * w0rt[1:2, :] + b0l[0:1, :])
    h = _layer_norm_rows(h0, g0, beta0)
    h_ref[...] = h
    cnt_ref[...] = cnt
    hp = jnp.maximum(
        jnp.dot(h, w1pt[...], preferred_element_type=_F32, precision=lax.Precision.HIGHEST) + b1p[0:1, :], 0.0)
    for g in range(NG):
        hpg_ref[g] = hp[:, g * FG:(g + 1) * FG]


def _tc1(partials, x, w0lt, w0rt, b0l, g0, beta0, w1pt, b1p):
    full = lambda r, c: pl.BlockSpec((r, c), lambda i: (0, 0))
    return pl.pallas_call(
        _tc1_body,
        grid=(N // NB,),
        in_specs=[
            pl.BlockSpec((2, NB, W0C), lambda i: (0, i, 0)),
            pl.BlockSpec((NB, 2), lambda i: (i, 0)),
            full(2, HID), full(2, HID), full(1, HID), full(1, HID),
            full(1, HID), full(HID, HID), full(1, HID),
        ],
        out_specs=[
            pl.BlockSpec((NB, HID), lambda i: (i, 0)),
            pl.BlockSpec((NG, NB, FG), lambda i: (0, i, 0)),
            pl.BlockSpec((NB, 1), lambda i: (i, 0)),
        ],
        out_shape=[
            jax.ShapeDtypeStruct((N, HID), _F32),
            jax.ShapeDtypeStruct((NG, ROWS, FG), _F32),
            jax.ShapeDtypeStruct((N, 1), _F32),
        ],
    )(partials, x, w0lt, w0rt, b0l, g0, beta0, w1pt, b1p)


def _tc2_body(sm_ref, cnt_ref, h_ref, w1lt, b1l, w1rt, g1, beta1,
              we1at, we1bt, ha_ref, hb_ref):
    summed = jnp.concatenate([sm_ref[g] for g in range(NG)], axis=1)
    mean1 = summed / cnt_ref[...]
    h1 = (jnp.dot(mean1, w1lt[...], preferred_element_type=_F32, precision=lax.Precision.HIGHEST)
          + jnp.dot(h_ref[...], w1rt[...], preferred_element_type=_F32, precision=lax.Precision.HIGHEST)
          + b1l[0:1, :])
    h2 = _layer_norm_rows(h1, g1, beta1)
    ha_ref[...] = jnp.dot(h2, we1at[...], preferred_element_type=_F32, precision=lax.Precision.HIGHEST)
    hb_ref[...] = jnp.dot(h2, we1bt[...], preferred_element_type=_F32, precision=lax.Precision.HIGHEST)


def _tc2(summed, cnt, h, w1lt, b1l, w1rt, g1, beta1, we1at, we1bt):
    full = lambda r, c: pl.BlockSpec((r, c), lambda i: (0, 0))
    return pl.pallas_call(
        _tc2_body,
        grid=(N // NB,),
        in_specs=[
            pl.BlockSpec((NG, NB, FG), lambda i: (0, i, 0)),
            pl.BlockSpec((NB, 1), lambda i: (i, 0)),
            pl.BlockSpec((NB, HID), lambda i: (i, 0)),
            full(HID, HID), full(1, HID), full(HID, HID), full(1, HID),
            full(1, HID), full(HID, HID), full(HID, HID),
        ],
        out_specs=[
            pl.BlockSpec((NB, HID), lambda i: (i, 0)),
            pl.BlockSpec((NB, HID), lambda i: (i, 0)),
        ],
        out_shape=[
            jax.ShapeDtypeStruct((N, HID), _F32),
            jax.ShapeDtypeStruct((N, HID), _F32),
        ],
    )(summed, cnt, h, w1lt, b1l, w1rt, g1, beta1, we1at, we1bt)


def _tc3_body(s_ref, be1, we2, be2, out_ref):
    t = jnp.maximum(s_ref[...] + be1[0:1, :], 0.0)
    out_ref[...] = jnp.sum(t * we2[0:1, :], axis=1, keepdims=True) + be2[0:1, :]


def _tc3(s, be1, we2, be2):
    full = lambda r, c: pl.BlockSpec((r, c), lambda i: (0, 0))
    return pl.pallas_call(
        _tc3_body,
        grid=(LPAD // LB,),
        in_specs=[
            pl.BlockSpec((LB, HID), lambda i: (i, 0)),
            full(1, HID), full(1, HID), full(1, 1),
        ],
        out_specs=pl.BlockSpec((LB, 1), lambda i: (i, 0)),
        out_shape=jax.ShapeDtypeStruct((LPAD, 1), _F32),
    )(s, be1, we2, be2)


def _agg_groups(table, acc, sb_s, sb_d, rows, gs, ss, grp, kk):
    def fire_g(g):
        q = g & 1
        return [pltpu.async_copy(table.at[sb_s.at[g * kk + r]],
                                 rows.at[q].at[r], gs[q]) for r in range(kk)]

    def fire_s(g):
        q = g & 1
        return [pltpu.async_copy(rows.at[q].at[r],
                                 acc.at[sb_d.at[g * kk + r]], ss[q], add=True)
                for r in range(kk)]

    scat = [None] * grp
    gcur = fire_g(0)
    for g in range(grp):
        for cp in gcur:
            cp.wait()
        if g + 1 < grp:
            if g >= 1:
                for cp in scat[g - 1]:
                    cp.wait()
            gcur = fire_g(g + 1)
        scat[g] = fire_s(g)
    for cp in scat[grp - 2]:
        cp.wait()
    for cp in scat[grp - 1]:
        cp.wait()


def _offset_slab(slab, off, sbr):
    def add_row(r, carry):
        for i in range(8):
            sl = pl.ds(i * 16, 16)
            slab[r, sl] = slab[r, sl] + off
        return carry

    lax.fori_loop(0, sbr, add_row, 0)


def _agg_pass(src_hbm, dst_hbm, table, acc, sblk, dblk, rows, isem, gs, ss,
              base, sbr, n_sb, grp, kk, off=None):
    pltpu.sync_copy(src_hbm.at[pl.ds(base, sbr)], sblk.at[0])
    pltpu.sync_copy(dst_hbm.at[pl.ds(base, sbr)], dblk.at[0])
    if off is not None:
        _offset_slab(sblk.at[0], off, sbr)

    def sb_pair(k, carry):
        for p in range(2):
            sb = 2 * k + p
            nxt = 1 - p
            nxt_row = base + jnp.where(sb + 1 < n_sb, (sb + 1) * sbr, 0)
            icps = [
                pltpu.async_copy(src_hbm.at[pl.ds(nxt_row, sbr)], sblk.at[nxt], isem),
                pltpu.async_copy(dst_hbm.at[pl.ds(nxt_row, sbr)], dblk.at[nxt], isem),
            ]
            _agg_groups(table, acc, sblk.at[p], dblk.at[p], rows, gs, ss, grp, kk)
            for cp in icps:
                cp.wait()
            if off is not None:
                _offset_slab(sblk.at[nxt], off, sbr)
        return carry

    lax.fori_loop(0, n_sb // 2, sb_pair, 0)


@functools.lru_cache(maxsize=None)
def _sc0_agg():
    @functools.partial(
        pl.kernel,
        out_type=jax.ShapeDtypeStruct((2, ROWS, W0C), _F32),
        mesh=_mesh(),
        compiler_params=_SC_PARAMS,
        scratch_types=[
            pltpu.VMEM((2, 20, 128), jnp.int32),
            pltpu.VMEM((2, 20, 128), jnp.int32),
            pltpu.VMEM((2, 2, 128, W0C), _F32),
            pltpu.VMEM_SHARED((ROWS, W0C), _F32),
            pltpu.VMEM_SHARED((TAB0, W0C), _F32),
            pltpu.SemaphoreType.DMA,
            pltpu.SemaphoreType.DMA,
            pltpu.SemaphoreType.DMA,
            pltpu.SemaphoreType.DMA,
            pltpu.SemaphoreType.DMA,
        ],
    )
    def body(src_hbm, dst_hbm, xp4_hbm, zeros_hbm, out_hbm,
             sblk, dblk, rows, acc, tab, isem, g0s, g1s, s0s, s1s):
        c = lax.axis_index("c")
        s = lax.axis_index("s")
        row0 = s * TPR
        pltpu.sync_copy(xp4_hbm.at[pl.ds(s * (TAB0 // 16), TAB0 // 16)],
                        tab.at[pl.ds(s * (TAB0 // 16), TAB0 // 16)])
        pltpu.sync_copy(zeros_hbm, acc.at[pl.ds(row0, TPR)])
        plsc.subcore_barrier()
        _agg_pass(src_hbm, dst_hbm, tab, acc, sblk, dblk, rows,
                  isem, (g0s, g1s), (s0s, s1s), c * 3200 + s * 200,
                  20, 10, 10, 2)
        plsc.subcore_barrier()
        pltpu.sync_copy(acc.at[pl.ds(row0, TPR)], out_hbm.at[c].at[pl.ds(row0, TPR)])

    return body


@functools.lru_cache(maxsize=None)
def _sc1_agg():
    @functools.partial(
        pl.kernel,
        out_type=jax.ShapeDtypeStruct((NG, ROWS, FG), _F32),
        mesh=_mesh(),
        compiler_params=_SC_PARAMS,
        scratch_types=[
            pltpu.VMEM((2, 20, 128), jnp.int32),
            pltpu.VMEM((2, 20, 128), jnp.int32),
            pltpu.VMEM((2, 2, 128, FG), _F32),
            pltpu.VMEM_SHARED((ROWS, FG), _F32),
            pltpu.SemaphoreType.DMA,
            pltpu.SemaphoreType.DMA,
            pltpu.SemaphoreType.DMA,
            pltpu.SemaphoreType.DMA,
            pltpu.SemaphoreType.DMA,
        ],
    )
    def body(src_hbm, dst_hbm, table_hbm, zeros_hbm, out_hbm,
             sblk, dblk, rows, acc, isem, g0s, g1s, s0s, s1s):
        c = lax.axis_index("c")
        s = lax.axis_index("s")
        row0 = s * TPR
        tab = table_hbm.at[c]

        def gi_body(gi, carry):
            g = (NG // 2) * c + gi
            pltpu.sync_copy(zeros_hbm, acc.at[pl.ds(row0, TPR)])
            plsc.subcore_barrier()
            _agg_pass(src_hbm, dst_hbm, tab, acc, sblk, dblk, rows,
                      isem, (g0s, g1s), (s0s, s1s), s * 400,
                      20, 20, 10, 2, off=gi * ROWS)
            plsc.subcore_barrier()
            pltpu.sync_copy(acc.at[pl.ds(row0, TPR)], out_hbm.at[g].at[pl.ds(row0, TPR)])
            plsc.subcore_barrier()
            return carry

        lax.fori_loop(0, NG // 2, gi_body, 0)

    return body


@functools.lru_cache(maxsize=None)
def _sc2_gather():
    @functools.partial(
        pl.kernel,
        out_type=jax.ShapeDtypeStruct((LPAD, HID), _F32),
        mesh=_mesh(),
        compiler_params=_SC_PARAMS,
        scratch_types=[
            pltpu.VMEM((25, 128), jnp.int32),
            pltpu.VMEM((25, 128), jnp.int32),
            pltpu.VMEM((2, 128, HID), _F32),
            pltpu.VMEM((2, 128, HID), _F32),
            pltpu.VMEM((2, 128, HID), _F32),
            pltpu.SemaphoreType.DMA,
            pltpu.SemaphoreType.DMA,
            pltpu.SemaphoreType.DMA,
            pltpu.SemaphoreType.DMA,
            pltpu.SemaphoreType.DMA,
            pltpu.SemaphoreType.DMA,
        ],
    )
    def body(ha_hbm, hb_hbm, sidx_hbm, didx_hbm, out_hbm,
             sblk, dblk, bufa, bufb, outb, sa0, sa1, sb0, sb1, w0, w1):
        c = lax.axis_index("c")
        s = lax.axis_index("s")
        base = (c * 16 + s) * 25
        pltpu.sync_copy(sidx_hbm.at[pl.ds(base, 25)], sblk)
        pltpu.sync_copy(didx_hbm.at[pl.ds(base, 25)], dblk)
        sems_a = (sa0, sa1)
        sems_b = (sb0, sb1)
        sems_w = (w0, w1)
        cpa = [None, None]
        cpb = [None, None]
        cpw = [None, None]
        cpa[0] = pltpu.async_copy(ha_hbm.at[sblk.at[0]], bufa.at[0], sems_a[0])
        cpb[0] = pltpu.async_copy(hb_hbm.at[dblk.at[0]], bufb.at[0], sems_b[0])
        for j in range(25):
            p = j & 1
            cpa[p].wait()
            cpb[p].wait()
            if j + 1 < 25:
                np_ = (j + 1) & 1
                cpa[np_] = pltpu.async_copy(ha_hbm.at[sblk.at[j + 1]], bufa.at[np_], sems_a[np_])
                cpb[np_] = pltpu.async_copy(hb_hbm.at[dblk.at[j + 1]], bufb.at[np_], sems_b[np_])
            if cpw[p] is not None:
                cpw[p].wait()

            def row_add(i, carry):
                for k in range(HID // 16):
                    sl = pl.ds(k * 16, 16)
                    outb[p, i, sl] = bufa[p, i, sl] + bufb[p, i, sl]
                return carry

            lax.fori_loop(0, 128, row_add, 0)
            cpw[p] = pltpu.async_copy(outb.at[p], out_hbm.at[pl.ds((base + j) * 128, 128)], sems_w[p])
        cpw[0].wait()
        cpw[1].wait()

    return body


def kernel(x, edge_index, edge_label_index, W0p, b0p, W0l, b0l, W0r, g0, beta0,
           W1p, b1p, W1l, b1l, W1r, g1, beta1, We1, be1, We2, be2):
    E = edge_index.shape[1]
    L = edge_label_index.shape[1]
    i32 = jnp.int32

    src = jnp.concatenate([edge_index[0], jnp.zeros((EPAD - E,), i32)])
    dst = jnp.concatenate([edge_index[1], jnp.full((EPAD - E,), N, i32)])
    src2d = src.reshape(EPAD // 128, 128)
    dst2d = dst.reshape(EPAD // 128, 128)
    sidx = jnp.concatenate([edge_label_index[0], jnp.zeros((LPAD - L,), i32)])
    didx = jnp.concatenate([edge_label_index[1], jnp.zeros((LPAD - L,), i32)])
    sidx2d = sidx.reshape(LPAD // 128, 128)
    didx2d = didx.reshape(LPAD // 128, 128)

    xp4 = _tc0(x, W0p, b0p.reshape(1, 2))
    partials = _sc0_agg()(src2d, dst2d, xp4, jnp.zeros((TPR, W0C), _F32))
    h, hpg, cnt = _tc1(
        partials, x, W0l.T, W0r.T, b0l.reshape(1, HID), g0.reshape(1, HID),
        beta0.reshape(1, HID), W1p.T, b1p.reshape(1, HID))
    summed1 = _sc1_agg()(src2d, dst2d, hpg.reshape(2, (NG // 2) * ROWS, FG),
                         jnp.zeros((TPR, FG), _F32))
    ha, hb = _tc2(
        summed1, cnt, h, W1l.T, b1l.reshape(1, HID), W1r.T, g1.reshape(1, HID),
        beta1.reshape(1, HID), We1[:, :HID].T, We1[:, HID:].T)
    s_feats = _sc2_gather()(ha, hb, sidx2d, didx2d)
    logits = _tc3(s_feats, be1.reshape(1, HID), We2.reshape(1, HID),
                  be2.reshape(1, 1))
    return logits[:L, 0]
